# Initial kernel scaffold; baseline (speedup 1.0000x reference)
import jax
import jax.numpy as jnp
from jax.experimental import pallas as pl


def kernel(positions, atom_features, lambda_sterics, lambda_electrostatics, retrieve_forces, batch, edge_index, gnn_edge_index, W1a, b1a, W1b, b1b, W2a, b2a, W2b, b2b, Ws1, bs1, Ws2, bs2, We1, be1, We2, be2):
    raise NotImplementedError("write your pallas kernel here")


# speedup vs baseline: 15.5773x; 15.5773x over previous
import functools

import jax
import jax.numpy as jnp
from jax import lax
from jax.experimental import pallas as pl
from jax.experimental.pallas import tpu as pltpu
from jax.experimental.pallas import tpu_sc as plsc

N = 10000
EG = 640000
EN = 320000
NB = 64
FRACTION = 0.5
GAMMA = 0.00542
OFFSET = 0.0195141
COUL = 138.935485
EPS_FAC = 1.0 - 1.0 / 78.5
K2 = -0.5 * COUL * EPS_FAC

NP = 10240
CS = 2000
C = 80
BE_G = 5120
BE_N = 3200
BN = 2000
NC = 2
NS = 16
NW = NC * NS


def _silu(x):
    return x * jax.nn.sigmoid(x)


def _dsilu(x):
    s = jax.nn.sigmoid(x)
    return s * (1.0 + x * (1.0 - s))


def _pcall(body, **kw):
    return pl.pallas_call(body, **kw)


def _sc_mesh():
    return plsc.VectorSubcoreMesh(core_axis_name="c", subcore_axis_name="s")


def _wid():
    return lax.axis_index("s") * NC + lax.axis_index("c")


def _sc_gather_scalars(tbl, idxA, rowsA, idxB, rowsB, E):
    T = tbl.shape[0]
    R = len(rowsA) + (len(rowsB) if rowsB is not None else 0)
    eper = E // NW
    nch = eper // CS

    @functools.partial(
        pl.kernel,
        mesh=_sc_mesh(),
        compiler_params=pltpu.CompilerParams(use_tc_tiling_on_sc=False, needs_layout_passes=False),
        out_type=jax.ShapeDtypeStruct((R, E), jnp.float32),
        scratch_types=[
            pltpu.VMEM((T * N,), jnp.float32),
            pltpu.VMEM((CS,), jnp.int32),
            pltpu.VMEM((CS,), jnp.int32),
            pltpu.VMEM((R, CS), jnp.float32),
        ],
    )
    def k(tbl_h, idxA_h, idxB_h, out_h, tbl_v, ia_v, ib_v, out_v):
        wid = _wid()
        pltpu.sync_copy(tbl_h, tbl_v)

        def chunk(j, carry):
            base = wid * eper + j * CS
            pltpu.sync_copy(idxA_h.at[pl.ds(base, CS)], ia_v)
            if rowsB is not None:
                pltpu.sync_copy(idxB_h.at[pl.ds(base, CS)], ib_v)

            def inner(i, carry2):
                off = i * 16
                iva = ia_v[pl.ds(off, 16)]
                r = 0
                for row in rowsA:
                    v = plsc.load_gather(tbl_v, [iva + row * N])
                    out_v[r, pl.ds(off, 16)] = v
                    r += 1
                if rowsB is not None:
                    ivb = ib_v[pl.ds(off, 16)]
                    for row in rowsB:
                        v = plsc.load_gather(tbl_v, [ivb + row * N])
                        out_v[r, pl.ds(off, 16)] = v
                        r += 1
                return carry2

            lax.fori_loop(0, CS // 16, inner, 0)
            pltpu.sync_copy(out_v, out_h.at[:, pl.ds(base, CS)])
            return carry

        lax.fori_loop(0, nch, chunk, 0)

    tblf = tbl.reshape(T * N)
    idxB_arg = idxB if idxB is not None else idxA
    return k(tblf, idxA, idxB_arg)


def _sc_scatter_scalars(vals, idxs, specs, E, nslots):
    eper = E // NW
    nch = eper // C
    ACC = nslots * NP
    stripe = ACC // NS

    @functools.partial(
        pl.kernel,
        mesh=_sc_mesh(),
        compiler_params=pltpu.CompilerParams(use_tc_tiling_on_sc=False, needs_layout_passes=False),
        out_type=jax.ShapeDtypeStruct((2, ACC), jnp.float32),
        scratch_types=[
            pltpu.VMEM_SHARED((ACC,), jnp.float32),
            pltpu.VMEM((stripe,), jnp.float32),
            pltpu.VMEM((len(idxs), C), jnp.int32),
            pltpu.VMEM((len(specs), C), jnp.int32),
            pltpu.VMEM((len(specs), C), jnp.float32),
        ],
    )
    def k(vals_h, *rest):
        idx_hs = rest[: len(idxs)]
        out_h = rest[len(idxs)]
        acc, zb, iv, io, vv = rest[len(idxs) + 1:]
        cc = lax.axis_index("c")
        sid = lax.axis_index("s")
        wid = _wid()

        def zloop(i, carry):
            zb[pl.ds(i * 16, 16)] = jnp.zeros((16,), jnp.float32)
            return carry

        lax.fori_loop(0, stripe // 16, zloop, 0)
        pltpu.sync_copy(zb, acc.at[pl.ds(sid * stripe, stripe)])
        plsc.subcore_barrier()

        def chunk(j, carry):
            cidx = wid * nch + j
            for t in range(len(idxs)):
                pltpu.sync_copy(idx_hs[t].at[cidx], iv.at[t])
            for si, (vr, ii, slot) in enumerate(specs):
                pltpu.sync_copy(vals_h.at[vr, pl.ds(cidx * C, C)], vv.at[si])

                def offl(i, carry2, _si=si, _ii=ii, _slot=slot):
                    io[_si, pl.ds(i * 16, 16)] = (
                        iv[_ii, pl.ds(i * 16, 16)] + _slot * NP)
                    return carry2

                lax.fori_loop(0, C // 16, offl, 0)
            for si in range(len(specs)):
                pltpu.sync_copy(vv.at[si], acc.at[io.at[si]], add=True)
            return carry

        lax.fori_loop(0, nch, chunk, 0)
        plsc.subcore_barrier()
        pltpu.sync_copy(acc.at[pl.ds(sid * stripe, stripe)],
                        out_h.at[cc, pl.ds(sid * stripe, stripe)])

    return k(vals, *idxs)


def _sc_scatter_rows(vals, idxA2, idxB2):
    eper = EN // NW
    nch = eper // C
    dual = idxB2 is not None
    nsets = 2 if dual else 1
    rstripe = N // NS

    scratch = [
        pltpu.VMEM_SHARED((N, 64), jnp.float32),
        pltpu.VMEM((rstripe // 5, 64), jnp.float32),
        pltpu.VMEM((C,), jnp.int32),
        pltpu.VMEM((C, 64), jnp.float32),
    ]
    if dual:
        scratch.insert(1, pltpu.VMEM_SHARED((N, 64), jnp.float32))

    @functools.partial(
        pl.kernel,
        mesh=_sc_mesh(),
        compiler_params=pltpu.CompilerParams(use_tc_tiling_on_sc=False, needs_layout_passes=False),
        out_type=jax.ShapeDtypeStruct((2, nsets, N, 64), jnp.float32),
        scratch_types=scratch,
    )
    def k(vals_h, idxA_h, idxB_h, out_h, *scr):
        if dual:
            accA, accB, zb, iv, vv = scr
            accs = (accA, accB)
            idx_hs = (idxA_h, idxB_h)
        else:
            accA, zb, iv, vv = scr
            accs = (accA,)
            idx_hs = (idxA_h,)
        cc = lax.axis_index("c")
        sid = lax.axis_index("s")
        wid = _wid()

        def zloop(i, carry):
            for c4 in range(4):
                zb[i, pl.ds(c4 * 16, 16)] = jnp.zeros((16,), jnp.float32)
            return carry

        lax.fori_loop(0, rstripe // 5, zloop, 0)
        for a in accs:
            for p in range(5):
                pltpu.sync_copy(
                    zb, a.at[pl.ds(sid * rstripe + p * (rstripe // 5),
                                   rstripe // 5), :])
        plsc.subcore_barrier()

        def chunk(j, carry):
            cidx = wid * nch + j
            pltpu.sync_copy(vals_h.at[pl.ds(cidx * C, C), :], vv)
            for t in range(nsets):
                pltpu.sync_copy(idx_hs[t].at[cidx], iv)
                pltpu.sync_copy(vv, accs[t].at[iv], add=True)
            return carry

        lax.fori_loop(0, nch, chunk, 0)
        plsc.subcore_barrier()
        for t in range(nsets):
            pltpu.sync_copy(accs[t].at[pl.ds(sid * rstripe, rstripe), :],
                            out_h.at[cc, t, pl.ds(sid * rstripe, rstripe), :])

    idxB_arg = idxB2 if dual else idxA2
    return k(vals, idxA2, idxB_arg)


def _sc_gather_rows_sum2(Ps, Pd, gs2, gd2):
    eper = EN // NW
    nch = eper // C

    @functools.partial(
        pl.kernel,
        mesh=_sc_mesh(),
        compiler_params=pltpu.CompilerParams(use_tc_tiling_on_sc=False, needs_layout_passes=False),
        out_type=jax.ShapeDtypeStruct((EN, 64), jnp.float32),
        scratch_types=[
            pltpu.VMEM((C,), jnp.int32),
            pltpu.VMEM((C,), jnp.int32),
            pltpu.VMEM((C, 64), jnp.float32),
            pltpu.VMEM((C, 64), jnp.float32),
            pltpu.SemaphoreType.DMA,
            pltpu.SemaphoreType.DMA,
        ],
    )
    def k(Ps_h, Pd_h, gs_h, gd_h, out_h, ia, ib, ra, rb, sa, sb):
        wid = _wid()

        def chunk(j, carry):
            cidx = wid * nch + j
            pltpu.sync_copy(gs_h.at[cidx], ia)
            pltpu.sync_copy(gd_h.at[cidx], ib)
            cpa = pltpu.async_copy(Ps_h.at[ia], ra, sa)
            cpb = pltpu.async_copy(Pd_h.at[ib], rb, sb)
            cpa.wait()
            cpb.wait()

            def addl(i, carry2):
                for c4 in range(4):
                    ra[i, pl.ds(c4 * 16, 16)] = (
                        ra[i, pl.ds(c4 * 16, 16)] + rb[i, pl.ds(c4 * 16, 16)])
                return carry2

            lax.fori_loop(0, C, addl, 0)
            pltpu.sync_copy(ra, out_h.at[pl.ds(cidx * C, C), :])
            return carry

        lax.fori_loop(0, nch, chunk, 0)

    return k(Ps, Pd, gs2, gd2)


def _sc_gather_rows(tab, idx2):
    eper = EN // NW
    nch = eper // C

    @functools.partial(
        pl.kernel,
        mesh=_sc_mesh(),
        compiler_params=pltpu.CompilerParams(use_tc_tiling_on_sc=False, needs_layout_passes=False),
        out_type=jax.ShapeDtypeStruct((EN, 64), jnp.float32),
        scratch_types=[
            pltpu.VMEM((C,), jnp.int32),
            pltpu.VMEM((C, 64), jnp.float32),
            pltpu.SemaphoreType.DMA,
        ],
    )
    def k(tab_h, idx_h, out_h, ia, ra, sa):
        wid = _wid()

        def chunk(j, carry):
            cidx = wid * nch + j
            pltpu.sync_copy(idx_h.at[cidx], ia)
            pltpu.async_copy(tab_h.at[ia], ra, sa).wait()
            pltpu.sync_copy(ra, out_h.at[pl.ds(cidx * C, C), :])
            return carry

        lax.fori_loop(0, nch, chunk, 0)

    return k(tab, idx2)


def _tc_node0(posT, afT, lamS, lamE, Ws1, bs1, Ws2T, bs2, We1, be1, We2T, be2):
    def body(posT_r, afT_r, lamS_r, lamE_r, Ws1_r, bs1_r, Ws2T_r, bs2_r,
             We1_r, be1_r, We2T_r, be2_r, gbt_r, gg_r):
        p = posT_r[...]
        af = afT_r[...]
        af0 = af[0:1]
        af1 = af[1:2]
        af2 = af[2:3]
        q = af0 - 0.5
        rho = 0.1 + 0.1 * af1
        scale = 0.8 + 0.4 * af2
        or_ = rho - OFFSET
        sa = GAMMA * (rho + 0.14) ** 2
        gbt_r[...] = jnp.concatenate([p, or_, scale, q, sa], axis=0)
        gsv = jax.nn.sigmoid(
            jnp.sum(_silu(lamS_r[...] * Ws1_r[...] + bs1_r[...]) * Ws2T_r[...],
                    axis=1, keepdims=True) + bs2_r[...])
        gev = jax.nn.sigmoid(
            jnp.sum(_silu(lamE_r[...] * We1_r[...] + be1_r[...]) * We2T_r[...],
                    axis=1, keepdims=True) + be2_r[...])
        gg_r[...] = jnp.concatenate([gsv, gev], axis=1)

    return _pcall(
        body,
        out_shape=(jax.ShapeDtypeStruct((7, N), jnp.float32),
                   jax.ShapeDtypeStruct((1, 2), jnp.float32)),
    )(posT, afT, lamS, lamE, Ws1, bs1, Ws2T, bs2, We1, be1, We2T, be2)


def _tc_gb1(g1):
    def body(g_r, d_r, I_r):
        g = g_r[...]
        ddx = g[0:1] - g[5:6]
        ddy = g[1:2] - g[6:7]
        ddz = g[2:3] - g[7:8]
        or_j = g[3:4]
        sc_j = g[4:5]
        or_i = g[8:9]
        d = jnp.sqrt(ddx * ddx + ddy * ddy + ddz * ddz + 1e-12)
        sr = sc_j * or_j
        L = jnp.maximum(jnp.abs(d - sr), or_i)
        U = d + sr
        I = 0.5 * (1.0 / L - 1.0 / U
                   + 0.25 * (d - sr * sr / d) * (1.0 / (U * U) - 1.0 / (L * L))
                   + 0.5 * jnp.log(L / U) / d)
        mask = (or_i < U).astype(jnp.float32)
        d_r[...] = d
        I_r[...] = I * mask

    nb = EG // BE_G
    return _pcall(
        body,
        grid=(nb,),
        in_specs=[pl.BlockSpec((9, BE_G), lambda i: (0, i))],
        out_specs=(pl.BlockSpec((1, BE_G), lambda i: (0, i)),
                   pl.BlockSpec((1, BE_G), lambda i: (0, i))),
        out_shape=(jax.ShapeDtypeStruct((1, EG), jnp.float32),
                   jax.ShapeDtypeStruct((1, EG), jnp.float32)),
    )(g1)


def _tc_node1(Ip, gbt):
    def body(Ip_r, gbt_r, B_r, dBdI_r):
        Isum = Ip_r[0, 0:1] + Ip_r[1, 0:1]
        or_ = gbt_r[3:4]
        x = 1.0 / or_ - Isum
        xc = jnp.clip(x, 0.5, 200.0)
        B = 1.0 / xc
        mask = ((x > 0.5) & (x < 200.0)).astype(jnp.float32)
        B_r[...] = B
        dBdI_r[...] = B * B * mask

    return _pcall(
        body,
        out_shape=(jax.ShapeDtypeStruct((1, N), jnp.float32),
                   jax.ShapeDtypeStruct((1, N), jnp.float32)),
    )(Ip, gbt)


def _tc_l1(X, W1a, b1a, W1b, b1b):
    def body(X_r, W1a_r, b1a_r, W1b_r, b1b_r, m_r):
        a1 = lax.dot_general(X_r[...], W1a_r[...], (((0,), (0,)), ((), ())),
                             preferred_element_type=jnp.float32) + b1a_r[...]
        m_r[...] = lax.dot_general(_silu(a1), W1b_r[...],
                                   (((1,), (0,)), ((), ())),
                                   preferred_element_type=jnp.float32) + b1b_r[...]

    nb = EN // BE_N
    return _pcall(
        body,
        grid=(nb,),
        in_specs=[
            pl.BlockSpec((10, BE_N), lambda i: (0, i)),
            pl.BlockSpec((10, 64), lambda i: (0, 0)),
            pl.BlockSpec((1, 64), lambda i: (0, 0)),
            pl.BlockSpec((64, 64), lambda i: (0, 0)),
            pl.BlockSpec((1, 64), lambda i: (0, 0)),
        ],
        out_specs=pl.BlockSpec((BE_N, 64), lambda i: (i, 0)),
        out_shape=jax.ShapeDtypeStruct((EN, 64), jnp.float32),
    )(X, W1a, b1a, W1b, b1b)


def _tc_h(hp, W2as, W2ad):
    def body(hp_r, W2as_r, W2ad_r, hpre_r, Ps_r, Pd_r):
        hpre = hp_r[0] + hp_r[1]
        h = _silu(hpre)
        hpre_r[...] = hpre
        Ps_r[...] = jnp.dot(h, W2as_r[...], preferred_element_type=jnp.float32)
        Pd_r[...] = jnp.dot(h, W2ad_r[...], preferred_element_type=jnp.float32)

    nb = N // BN
    return _pcall(
        body,
        grid=(nb,),
        in_specs=[
            pl.BlockSpec((2, BN, 64), lambda i: (0, i, 0)),
            pl.BlockSpec((64, 64), lambda i: (0, 0)),
            pl.BlockSpec((64, 64), lambda i: (0, 0)),
        ],
        out_specs=(pl.BlockSpec((BN, 64), lambda i: (i, 0)),
                   pl.BlockSpec((BN, 64), lambda i: (i, 0)),
                   pl.BlockSpec((BN, 64), lambda i: (i, 0))),
        out_shape=(jax.ShapeDtypeStruct((N, 64), jnp.float32),
                   jax.ShapeDtypeStruct((N, 64), jnp.float32),
                   jax.ShapeDtypeStruct((N, 64), jnp.float32)),
    )(hp, W2as, W2ad)


def _tc_l2(Asum, b2a, W2b):
    def body(A_r, b2a_r, W2b_r, m2_r):
        sa2 = _silu(A_r[...] + b2a_r[...])
        m2_r[...] = lax.dot_general(W2b_r[...], sa2, (((0,), (1,)), ((), ())),
                                    preferred_element_type=jnp.float32)

    nb = EN // BE_N
    return _pcall(
        body,
        grid=(nb,),
        in_specs=[
            pl.BlockSpec((BE_N, 64), lambda i: (i, 0)),
            pl.BlockSpec((1, 64), lambda i: (0, 0)),
            pl.BlockSpec((64, 2), lambda i: (0, 0)),
        ],
        out_specs=pl.BlockSpec((2, BE_N), lambda i: (0, i)),
        out_shape=jax.ShapeDtypeStruct((2, EN), jnp.float32),
    )(Asum, b2a, W2b)


def _tc_node2(cp, B, gbt, gg):
    def body(cp_r, B_r, gbt_r, gg_r, Bc_r, parte_r, gcp1_r, s0_r, c0_r):
        cpre0 = cp_r[0, 0:1] + cp_r[1, 0:1]
        cpre1 = cp_r[0, 1:2] + cp_r[1, 1:2]
        c0 = jax.nn.sigmoid(cpre0)
        c1 = jax.nn.sigmoid(cpre1)
        B = B_r[...]
        q = gbt_r[5:6]
        sa = gbt_r[6:7]
        gsv = gg_r[0:1, 0:1]
        gev = gg_r[0:1, 1:2]
        Bc = B * (FRACTION * c0 + (1.0 - FRACTION))
        e_self = K2 * q * q / Bc
        Bc_r[...] = Bc
        parte_r[...] = e_self * gev + sa * c1 * gsv
        gcp1_r[...] = sa * gsv * c1 * (1.0 - c1)
        s0_r[...] = c0 * (1.0 - c0)
        c0_r[...] = c0

    shp = jax.ShapeDtypeStruct((1, N), jnp.float32)
    return _pcall(
        body,
        out_shape=(shp, shp, shp, shp, shp),
    )(cp, B, gbt, gg)


def _tc_gb2(g2, d, gg):
    def body(g2_r, d_r, gg_r, out_r):
        q_s = g2_r[0:1]
        Bc_s = g2_r[1:2]
        q_d = g2_r[2:3]
        Bc_d = g2_r[3:4]
        d = d_r[...]
        gev = gg_r[0:1, 1:2]
        u = Bc_d * Bc_s
        ex = jnp.exp(-(d * d) / (4.0 * u))
        f2 = d * d + u * ex
        f = jnp.sqrt(f2)
        w = K2 * q_d * q_s
        e_pair = w / f
        dedf = -w / f2
        dfdd = (2.0 * d - 0.5 * d * ex) / (2.0 * f)
        dfdu = ex * (1.0 + d * d / (4.0 * u)) / (2.0 * f)
        gdd = gev * dedf * dfdd
        gu = gev * dedf * dfdu
        out_r[...] = jnp.concatenate(
            [e_pair, gu * Bc_s, gu * Bc_d, gdd], axis=0)

    nb = EG // BE_G
    return _pcall(
        body,
        grid=(nb,),
        in_specs=[
            pl.BlockSpec((4, BE_G), lambda i: (0, i)),
            pl.BlockSpec((1, BE_G), lambda i: (0, i)),
            pl.BlockSpec((1, 2), lambda i: (0, 0)),
        ],
        out_specs=pl.BlockSpec((4, BE_G), lambda i: (0, i)),
        out_shape=jax.ShapeDtypeStruct((4, EG), jnp.float32),
    )(g2, d, gg)


def _tc_node3b(ep, parte, gbt, Bc, B, s0, c0, gcp1, gg):
    def body(ep_r, parte_r, gbt_r, Bc_r, B_r, s0_r, c0_r, gcp1_r, gg_r,
             ea_r, gcp_r, gBdir_r):
        e_gb = ep_r[0, 0:1] + ep_r[1, 0:1]
        gBi_n = ep_r[0, 1:2] + ep_r[1, 1:2]
        gBj_n = ep_r[0, 2:3] + ep_r[1, 2:3]
        q = gbt_r[5:6]
        gev = gg_r[0:1, 1:2]
        Bc = Bc_r[...]
        B = B_r[...]
        s0 = s0_r[...]
        c0 = c0_r[...]
        ea_r[...] = parte_r[...] + e_gb * gev
        gBc = gBi_n + gBj_n - gev * K2 * q * q / (Bc * Bc)
        gcp0 = gBc * B * FRACTION * s0
        gcp_r[...] = jnp.concatenate([gcp0, gcp1_r[...]], axis=0)
        gBdir_r[...] = gBc * (FRACTION * c0 + (1.0 - FRACTION))

    return _pcall(
        body,
        out_shape=(jax.ShapeDtypeStruct((1, N), jnp.float32),
                   jax.ShapeDtypeStruct((2, N), jnp.float32),
                   jax.ShapeDtypeStruct((1, N), jnp.float32)),
    )(ep, parte, gbt, Bc, B, s0, c0, gcp1, gg)


def _tc_l2b(Asum, gcpg, b2a, W2b):
    def body(A_r, gcpg_r, b2a_r, W2b_r, ga2_r):
        a2 = A_r[...] + b2a_r[...]
        gsa2 = lax.dot_general(gcpg_r[...], W2b_r[...], (((0,), (1,)), ((), ())),
                               preferred_element_type=jnp.float32)
        ga2_r[...] = gsa2 * _dsilu(a2)

    nb = EN // BE_N
    return _pcall(
        body,
        grid=(nb,),
        in_specs=[
            pl.BlockSpec((BE_N, 64), lambda i: (i, 0)),
            pl.BlockSpec((2, BE_N), lambda i: (0, i)),
            pl.BlockSpec((1, 64), lambda i: (0, 0)),
            pl.BlockSpec((64, 2), lambda i: (0, 0)),
        ],
        out_specs=pl.BlockSpec((BE_N, 64), lambda i: (i, 0)),
        out_shape=jax.ShapeDtypeStruct((EN, 64), jnp.float32),
    )(Asum, gcpg, b2a, W2b)


def _tc_l1bn(Gp, hpre, W2asT, W2adT, W1bT):
    def body(Gp_r, hpre_r, W2asT_r, W2adT_r, W1bT_r, R_r):
        Gs = Gp_r[0, 0] + Gp_r[1, 0]
        Gd = Gp_r[0, 1] + Gp_r[1, 1]
        gh = (jnp.dot(Gs, W2asT_r[...], preferred_element_type=jnp.float32)
              + jnp.dot(Gd, W2adT_r[...], preferred_element_type=jnp.float32))
        ghp = gh * _dsilu(hpre_r[...])
        R_r[...] = jnp.dot(ghp, W1bT_r[...], preferred_element_type=jnp.float32)

    nb = N // BN
    return _pcall(
        body,
        grid=(nb,),
        in_specs=[
            pl.BlockSpec((2, 2, BN, 64), lambda i: (0, 0, i, 0)),
            pl.BlockSpec((BN, 64), lambda i: (i, 0)),
            pl.BlockSpec((64, 64), lambda i: (0, 0)),
            pl.BlockSpec((64, 64), lambda i: (0, 0)),
            pl.BlockSpec((64, 64), lambda i: (0, 0)),
        ],
        out_specs=pl.BlockSpec((BN, 64), lambda i: (i, 0)),
        out_shape=jax.ShapeDtypeStruct((N, 64), jnp.float32),
    )(Gp, hpre, W2asT, W2adT, W1bT)


def _tc_l1b(X, Rg, W1a, b1a, Wb):
    def body(X_r, Rg_r, W1a_r, b1a_r, Wb_r, gB_r):
        a1 = lax.dot_general(X_r[...], W1a_r[...], (((0,), (0,)), ((), ())),
                             preferred_element_type=jnp.float32) + b1a_r[...]
        ga1 = Rg_r[...] * _dsilu(a1)
        gB_r[...] = lax.dot_general(Wb_r[...], ga1, (((1,), (1,)), ((), ())),
                                    preferred_element_type=jnp.float32)

    nb = EN // BE_N
    return _pcall(
        body,
        grid=(nb,),
        in_specs=[
            pl.BlockSpec((10, BE_N), lambda i: (0, i)),
            pl.BlockSpec((BE_N, 64), lambda i: (i, 0)),
            pl.BlockSpec((10, 64), lambda i: (0, 0)),
            pl.BlockSpec((1, 64), lambda i: (0, 0)),
            pl.BlockSpec((2, 64), lambda i: (0, 0)),
        ],
        out_specs=pl.BlockSpec((2, BE_N), lambda i: (0, i)),
        out_shape=jax.ShapeDtypeStruct((2, EN), jnp.float32),
    )(X, Rg, W1a, b1a, Wb)


def _tc_node4(gp, gBdir, dBdI):
    def body(gp_r, gBdir_r, dBdI_r, gI_r):
        gB = gp_r[0, 0:1] + gp_r[1, 0:1] + gp_r[0, 1:2] + gp_r[1, 1:2]
        gI_r[...] = (gBdir_r[...] + gB) * dBdI_r[...]

    return _pcall(
        body,
        out_shape=jax.ShapeDtypeStruct((1, N), jnp.float32),
    )(gp, gBdir, dBdI)


def _tc_force(g1, d, gbo, gI):
    def body(g_r, d_r, gbo_r, gI_r, fv_r):
        g = g_r[...]
        ddx = g[0:1] - g[5:6]
        ddy = g[1:2] - g[6:7]
        ddz = g[2:3] - g[7:8]
        or_j = g[3:4]
        sc_j = g[4:5]
        or_i = g[8:9]
        d = d_r[...]
        sr = sc_j * or_j
        L = jnp.maximum(jnp.abs(d - sr), or_i)
        U = d + sr
        mask = (or_i < U).astype(jnp.float32)
        absds = jnp.abs(d - sr)
        dLdd = jnp.sign(d - sr) * (absds > or_i).astype(jnp.float32)
        iL = 1.0 / L
        iU = 1.0 / U
        idd = 1.0 / d
        t = d - sr * sr * idd
        dIdL = 0.5 * (-iL * iL + 0.5 * t * iL * iL * iL + 0.5 * iL * idd)
        dIdU = 0.5 * (iU * iU - 0.5 * t * iU * iU * iU - 0.5 * iU * idd)
        dIdd_exp = 0.5 * (0.25 * (1.0 + (sr * idd) ** 2) * (iU * iU - iL * iL)
                          - 0.5 * jnp.log(L * iU) * idd * idd)
        dIdd = (dIdL * dLdd + dIdU + dIdd_exp) * mask
        g_tot = gbo_r[3:4] + gI_r[...] * dIdd
        coef = g_tot * idd
        fv_r[...] = jnp.concatenate([coef * ddx, coef * ddy, coef * ddz],
                                    axis=0)

    nb = EG // BE_G
    return _pcall(
        body,
        grid=(nb,),
        in_specs=[
            pl.BlockSpec((9, BE_G), lambda i: (0, i)),
            pl.BlockSpec((1, BE_G), lambda i: (0, i)),
            pl.BlockSpec((4, BE_G), lambda i: (0, i)),
            pl.BlockSpec((1, BE_G), lambda i: (0, i)),
        ],
        out_specs=pl.BlockSpec((3, BE_G), lambda i: (0, i)),
        out_shape=jax.ShapeDtypeStruct((3, EG), jnp.float32),
    )(g1, d, gbo, gI)


def _tc_final(fp, e_atom, batT):
    def body(fp_r, ea_r, bat_r, F_r, en_r):
        F_r[...] = ((fp_r[0, 0:3] + fp_r[1, 0:3])
                    - (fp_r[0, 3:6] + fp_r[1, 3:6]))
        oh = (bat_r[...] == lax.broadcasted_iota(jnp.int32, (N, NB), 1)
              ).astype(jnp.float32)
        en_r[...] = lax.dot_general(ea_r[...], oh, (((1,), (0,)), ((), ())),
                                    preferred_element_type=jnp.float32)

    return _pcall(
        body,
        out_shape=(jax.ShapeDtypeStruct((3, N), jnp.float32),
                   jax.ShapeDtypeStruct((1, NB), jnp.float32)),
    )(fp, e_atom, batT)


def kernel(positions, atom_features, lambda_sterics, lambda_electrostatics,
           retrieve_forces, batch, edge_index, gnn_edge_index,
           W1a, b1a, W1b, b1b, W2a, b2a, W2b, b2b,
           Ws1, bs1, Ws2, bs2, We1, be1, We2, be2):
    f32 = jnp.float32
    posT = positions.T.astype(f32)
    afT = atom_features.T.astype(f32)
    src = edge_index[0].astype(jnp.int32)
    dst = edge_index[1].astype(jnp.int32)
    gs = gnn_edge_index[0].astype(jnp.int32)
    gd = gnn_edge_index[1].astype(jnp.int32)
    src2 = src.reshape(EG // C, C)
    dst2 = dst.reshape(EG // C, C)
    gs2 = gs.reshape(EN // C, C)
    gd2 = gd.reshape(EN // C, C)
    batT = batch.astype(jnp.int32).reshape(N, 1)

    gbt, gg = _tc_node0(
        posT, afT,
        lambda_sterics.reshape(1, 1), lambda_electrostatics.reshape(1, 1),
        Ws1, bs1.reshape(1, 32), Ws2.reshape(1, 32), bs2.reshape(1, 1),
        We1, be1.reshape(1, 32), We2.reshape(1, 32), be2.reshape(1, 1))

    g1 = _sc_gather_scalars(gbt, src, (0, 1, 2, 3, 4), dst, (0, 1, 2, 3), EG)
    d, I = _tc_gb1(g1)
    IpF = _sc_scatter_scalars(I, [dst2], [(0, 0, 0)], EG, 1)
    Ip = IpF.reshape(2, 1, NP)[:, :, :N]
    B, dBdI = _tc_node1(Ip, gbt)

    gnt = jnp.concatenate([B, afT[:4]], axis=0)
    X = _sc_gather_scalars(gnt, gs, (0, 1, 2, 3, 4), gd, (0, 1, 2, 3, 4), EN)
    m = _tc_l1(X, W1a, b1a.reshape(1, 64), W1b, b1b.reshape(1, 64))
    hp = _sc_scatter_rows(m, gd2, None)
    hpre, Ps, Pd = _tc_h(hp[:, 0], W2a[:64], W2a[64:])

    Asum = _sc_gather_rows_sum2(Ps, Pd, gs2, gd2)
    m2 = _tc_l2(Asum, b2a.reshape(1, 64), W2b)
    cpF = _sc_scatter_scalars(m2, [gd2], [(0, 0, 0), (1, 0, 1)], EN, 2)
    cp = cpF.reshape(2, 2, NP)[:, :, :N]
    Bc, parte, gcp1, s0, c0 = _tc_node2(cp, B, gbt, gg)

    qBc = jnp.concatenate([gbt[5:6], Bc], axis=0)
    g2 = _sc_gather_scalars(qBc, src, (0, 1), dst, (0, 1), EG)
    gbo = _tc_gb2(g2, d, gg)
    epF = _sc_scatter_scalars(gbo, [dst2, src2],
                              [(0, 0, 0), (1, 0, 1), (2, 1, 2)], EG, 3)
    ep = epF.reshape(2, 3, NP)[:, :, :N]
    e_atom, gcp, gBdir = _tc_node3b(ep, parte, gbt, Bc, B, s0, c0, gcp1, gg)

    gcpg = _sc_gather_scalars(gcp, gd, (0, 1), None, None, EN)
    ga2 = _tc_l2b(Asum, gcpg, b2a.reshape(1, 64), W2b)
    Gp = _sc_scatter_rows(ga2, gs2, gd2)
    Rm = _tc_l1bn(Gp, hpre, W2a[:64].T, W2a[64:].T, W1b.T)
    Rg = _sc_gather_rows(Rm, gd2)
    Wb = jnp.concatenate([W1a[0:1], W1a[5:6]], axis=0)
    gB2 = _tc_l1b(X, Rg, W1a, b1a.reshape(1, 64), Wb)
    gpF = _sc_scatter_scalars(gB2, [gs2, gd2], [(0, 0, 0), (1, 1, 1)], EN, 2)
    gp = gpF.reshape(2, 2, NP)[:, :, :N]
    gIsum = _tc_node4(gp, gBdir, dBdI)

    gI = _sc_gather_scalars(gIsum, dst, (0,), None, None, EG)
    fv = _tc_force(g1, d, gbo, gI)
    fpF = _sc_scatter_scalars(
        fv, [dst2, src2],
        [(0, 0, 0), (1, 0, 1), (2, 0, 2), (0, 1, 3), (1, 1, 4), (2, 1, 5)],
        EG, 6)
    fp = fpF.reshape(2, 6, NP)[:, :, :N]
    F, en = _tc_final(fp, e_atom, batT)

    energy = en.reshape(NB, 1)
    forces = F.T
    return energy, forces


# speedup vs baseline: 40.4192x; 2.5948x over previous
import functools

import jax
import jax.numpy as jnp
from jax import lax
from jax.experimental import pallas as pl
from jax.experimental.pallas import tpu as pltpu
from jax.experimental.pallas import tpu_sc as plsc

N = 10000
EG = 640000
EN = 320000
NB = 64
FRACTION = 0.5
GAMMA = 0.00542
OFFSET = 0.0195141
COUL = 138.935485
EPS_FAC = 1.0 - 1.0 / 78.5
K2 = -0.5 * COUL * EPS_FAC

NP = 10240
CS = 2000
C = 80
BE_G = 5120
BE_N = 4096
BN = 2000
NC = 2
NS = 16
NW = NC * NS


def _silu(x):
    return x * jax.nn.sigmoid(x)


def _dsilu(x):
    s = jax.nn.sigmoid(x)
    return s * (1.0 + x * (1.0 - s))


def _pcall(body, **kw):
    return pl.pallas_call(body, **kw)


_SC_PARAMS = dict(
    compiler_params=pltpu.CompilerParams(use_tc_tiling_on_sc=False,
                                         needs_layout_passes=False))


def _sc_mesh():
    return plsc.VectorSubcoreMesh(core_axis_name="c", subcore_axis_name="s")


def _wid():
    return lax.axis_index("s") * NC + lax.axis_index("c")


def _sc_gather_scalars(tbl, idxA, rowsA, idxB, rowsB, E):
    T = tbl.shape[0] // N
    R = len(rowsA) + (len(rowsB) if rowsB is not None else 0)
    eper = E // NW
    nch = eper // CS

    @functools.partial(
        pl.kernel,
        mesh=_sc_mesh(),
        out_type=tuple(jax.ShapeDtypeStruct((E,), jnp.float32)
                       for _ in range(R)),
        scratch_types=[
            pltpu.VMEM((T * N,), jnp.float32),
            pltpu.VMEM((2, CS), jnp.int32),
            pltpu.VMEM((2, CS), jnp.int32),
            pltpu.VMEM((2, R, CS), jnp.float32),
            pltpu.SemaphoreType.DMA,
            pltpu.SemaphoreType.DMA,
            pltpu.SemaphoreType.DMA,
        ],
        **_SC_PARAMS,
    )
    def k(tbl_h, idxA_h, idxB_h, *rest):
        out_hs = rest[:R]
        tbl_v, ia_v, ib_v, out_v, s_t, s_l, s_o = rest[R:]
        wid = _wid()
        tcp = pltpu.async_copy(tbl_h, tbl_v, s_t)

        def load(b, buf):
            base = wid * eper + b * CS
            h = [pltpu.async_copy(idxA_h.at[pl.ds(base, CS)], ia_v.at[buf],
                                  s_l)]
            if rowsB is not None:
                h.append(pltpu.async_copy(idxB_h.at[pl.ds(base, CS)],
                                          ib_v.at[buf], s_l))
            return h

        pend_l = load(0, 0)
        tcp.wait()
        pend_o = [[], []]
        for b in range(nch):
            buf = b & 1
            for h in pend_l:
                h.wait()
            if b + 1 < nch:
                pend_l = load(b + 1, buf ^ 1)
            for h in pend_o[buf]:
                h.wait()

            def inner(i, carry, _buf=buf):
                for u in range(5):
                    off = i * 80 + u * 16
                    iva = ia_v[_buf, pl.ds(off, 16)]
                    r = 0
                    for row in rowsA:
                        v = plsc.load_gather(tbl_v, [iva + row * N])
                        out_v[_buf, r, pl.ds(off, 16)] = v
                        r += 1
                    if rowsB is not None:
                        ivb = ib_v[_buf, pl.ds(off, 16)]
                        for row in rowsB:
                            v = plsc.load_gather(tbl_v, [ivb + row * N])
                            out_v[_buf, r, pl.ds(off, 16)] = v
                            r += 1
                return carry

            lax.fori_loop(0, CS // 80, inner, 0)
            base = wid * eper + b * CS
            pend_o[buf] = [
                pltpu.async_copy(out_v.at[buf, r],
                                 out_hs[r].at[pl.ds(base, CS)], s_o)
                for r in range(R)]
        for hb in pend_o:
            for h in hb:
                h.wait()

    idxB_arg = idxB if idxB is not None else idxA
    return k(tbl, idxA, idxB_arg)


def _sc_scatter_scalars(vals, idxs, specs, E, nslots):
    eper = E // NW
    BK = 400
    W = BK // C
    nblk = eper // BK
    ACC = nslots * NP
    stripe = ACC // NS
    nv = len(vals)
    ni = len(idxs)
    ns = len(specs)

    @functools.partial(
        pl.kernel,
        mesh=_sc_mesh(),
        out_type=jax.ShapeDtypeStruct((2, ACC), jnp.float32),
        scratch_types=[
            pltpu.VMEM_SHARED((ACC,), jnp.float32),
            pltpu.VMEM((stripe,), jnp.float32),
            pltpu.VMEM((ni, W, C), jnp.int32),
            pltpu.VMEM((ns, W, C), jnp.int32),
            pltpu.VMEM((ns, BK), jnp.float32),
            pltpu.SemaphoreType.DMA,
            pltpu.SemaphoreType.DMA,
        ],
        **_SC_PARAMS,
    )
    def k(*args):
        vals_hs = args[:nv]
        idx_hs = args[nv:nv + ni]
        out_h = args[nv + ni]
        acc, zb, iv, io, vv, s_l, s_s = args[nv + ni + 1:]
        cc = lax.axis_index("c")
        sid = lax.axis_index("s")
        wid = _wid()

        def zloop(i, carry):
            zb[pl.ds(i * 16, 16)] = jnp.zeros((16,), jnp.float32)
            return carry

        lax.fori_loop(0, stripe // 16, zloop, 0)
        pltpu.sync_copy(zb, acc.at[pl.ds(sid * stripe, stripe)])
        plsc.subcore_barrier()

        def blk(b, carry):
            irow = wid * (eper // C) + b * W
            hs = []
            for t in range(ni):
                hs.append(pltpu.async_copy(idx_hs[t].at[pl.ds(irow, W)],
                                           iv.at[t], s_l))
            for si, (vr, ii, slot) in enumerate(specs):
                hs.append(pltpu.async_copy(
                    vals_hs[vr].at[pl.ds(wid * eper + b * BK, BK)],
                    vv.at[si], s_l))
            for h in hs:
                h.wait()
            for si, (vr, ii, slot) in enumerate(specs):
                def offl(i, carry2, _si=si, _ii=ii, _slot=slot):
                    for w in range(W):
                        io[_si, w, pl.ds(i * 16, 16)] = (
                            iv[_ii, w, pl.ds(i * 16, 16)] + _slot * NP)
                    return carry2

                lax.fori_loop(0, C // 16, offl, 0)
            ss = []
            for si in range(ns):
                for w in range(W):
                    ss.append(pltpu.async_copy(
                        vv.at[si, pl.ds(w * C, C)], acc.at[io.at[si, w]],
                        s_s, add=True))
            for h in ss:
                h.wait()
            return carry

        lax.fori_loop(0, nblk, blk, 0)
        plsc.subcore_barrier()
        pltpu.sync_copy(acc.at[pl.ds(sid * stripe, stripe)],
                        out_h.at[cc, pl.ds(sid * stripe, stripe)])

    return k(*vals, *idxs)


def _sc_scatter_rows(vals, idxA2, idxB2):
    eper = EN // NW
    BK = 400
    W = BK // C
    nblk = eper // BK
    dual = idxB2 is not None
    nsets = 2 if dual else 1
    rstripe = N // NS

    scratch = [
        pltpu.VMEM_SHARED((N, 64), jnp.float32),
        pltpu.VMEM((rstripe // 5, 64), jnp.float32),
        pltpu.VMEM((nsets, W, C), jnp.int32),
        pltpu.VMEM((BK, 64), jnp.float32),
        pltpu.SemaphoreType.DMA,
        pltpu.SemaphoreType.DMA,
    ]
    if dual:
        scratch.insert(1, pltpu.VMEM_SHARED((N, 64), jnp.float32))

    @functools.partial(
        pl.kernel,
        mesh=_sc_mesh(),
        out_type=jax.ShapeDtypeStruct((2, nsets, N, 64), jnp.float32),
        scratch_types=scratch,
        **_SC_PARAMS,
    )
    def k(vals_h, idxA_h, idxB_h, out_h, *scr):
        if dual:
            accA, accB, zb, iv, vv, s_l, s_s = scr
            accs = (accA, accB)
            idx_hs = (idxA_h, idxB_h)
        else:
            accA, zb, iv, vv, s_l, s_s = scr
            accs = (accA,)
            idx_hs = (idxA_h,)
        cc = lax.axis_index("c")
        sid = lax.axis_index("s")
        wid = _wid()

        def zloop(i, carry):
            for c4 in range(4):
                zb[i, pl.ds(c4 * 16, 16)] = jnp.zeros((16,), jnp.float32)
            return carry

        lax.fori_loop(0, rstripe // 5, zloop, 0)
        for a in accs:
            for p in range(5):
                pltpu.sync_copy(
                    zb, a.at[pl.ds(sid * rstripe + p * (rstripe // 5),
                                   rstripe // 5), :])
        plsc.subcore_barrier()

        def blk(b, carry):
            irow = wid * (eper // C) + b * W
            hs = [pltpu.async_copy(vals_h.at[pl.ds(wid * eper + b * BK, BK), :],
                                   vv, s_l)]
            for t in range(nsets):
                hs.append(pltpu.async_copy(idx_hs[t].at[pl.ds(irow, W)],
                                           iv.at[t], s_l))
            for h in hs:
                h.wait()
            ss = []
            for t in range(nsets):
                for w in range(W):
                    ss.append(pltpu.async_copy(
                        vv.at[pl.ds(w * C, C), :], accs[t].at[iv.at[t, w]],
                        s_s, add=True))
            for h in ss:
                h.wait()
            return carry

        lax.fori_loop(0, nblk, blk, 0)
        plsc.subcore_barrier()
        for t in range(nsets):
            pltpu.sync_copy(accs[t].at[pl.ds(sid * rstripe, rstripe), :],
                            out_h.at[cc, t, pl.ds(sid * rstripe, rstripe), :])

    idxB_arg = idxB2 if dual else idxA2
    return k(vals, idxA2, idxB_arg)


def _sc_gather_rows_sum2(Ps, Pd, gs2, gd2):
    eper = EN // NW
    BK = 400
    W = BK // C
    nblk = eper // BK

    @functools.partial(
        pl.kernel,
        mesh=_sc_mesh(),
        out_type=jax.ShapeDtypeStruct((EN, 64), jnp.float32),
        scratch_types=[
            pltpu.VMEM((2, W, C), jnp.int32),
            pltpu.VMEM((2, BK, 64), jnp.float32),
            pltpu.VMEM((2, BK, 64), jnp.float32),
            pltpu.SemaphoreType.DMA,
            pltpu.SemaphoreType.DMA,
            pltpu.SemaphoreType.DMA,
        ],
        **_SC_PARAMS,
    )
    def k(Ps_h, Pd_h, gs_h, gd_h, out_h, iv, ra, rb, s_l, s_g, s_o):
        wid = _wid()
        pend_o = [None, None]
        for b in range(nblk):
            buf = b & 1
            irow = wid * (eper // C) + b * W
            h1 = pltpu.async_copy(gs_h.at[pl.ds(irow, W)], iv.at[0], s_l)
            h2 = pltpu.async_copy(gd_h.at[pl.ds(irow, W)], iv.at[1], s_l)
            h1.wait()
            h2.wait()
            if pend_o[buf] is not None:
                pend_o[buf].wait()
            gh = []
            for w in range(W):
                gh.append(pltpu.async_copy(
                    Ps_h.at[iv.at[0, w]],
                    ra.at[buf, pl.ds(w * C, C), :], s_g))
                gh.append(pltpu.async_copy(
                    Pd_h.at[iv.at[1, w]],
                    rb.at[buf, pl.ds(w * C, C), :], s_g))
            for h in gh:
                h.wait()

            def addl(i, carry2, _buf=buf):
                for c4 in range(4):
                    ra[_buf, i, pl.ds(c4 * 16, 16)] = (
                        ra[_buf, i, pl.ds(c4 * 16, 16)]
                        + rb[_buf, i, pl.ds(c4 * 16, 16)])
                return carry2

            lax.fori_loop(0, BK, addl, 0)
            pend_o[buf] = pltpu.async_copy(
                ra.at[buf], out_h.at[pl.ds(wid * eper + b * BK, BK), :], s_o)
        for h in pend_o:
            if h is not None:
                h.wait()

    return k(Ps, Pd, gs2, gd2)


def _sc_gather_rows(tab, idx2):
    eper = EN // NW
    BK = 400
    W = BK // C
    nblk = eper // BK

    @functools.partial(
        pl.kernel,
        mesh=_sc_mesh(),
        out_type=jax.ShapeDtypeStruct((EN, 64), jnp.float32),
        scratch_types=[
            pltpu.VMEM((W, C), jnp.int32),
            pltpu.VMEM((2, BK, 64), jnp.float32),
            pltpu.SemaphoreType.DMA,
            pltpu.SemaphoreType.DMA,
            pltpu.SemaphoreType.DMA,
        ],
        **_SC_PARAMS,
    )
    def k(tab_h, idx_h, out_h, iv, ra, s_l, s_g, s_o):
        wid = _wid()
        pend_o = [None, None]
        for b in range(nblk):
            buf = b & 1
            irow = wid * (eper // C) + b * W
            pltpu.async_copy(idx_h.at[pl.ds(irow, W)], iv, s_l).wait()
            if pend_o[buf] is not None:
                pend_o[buf].wait()
            gh = []
            for w in range(W):
                gh.append(pltpu.async_copy(
                    tab_h.at[iv.at[w]], ra.at[buf, pl.ds(w * C, C), :], s_g))
            for h in gh:
                h.wait()
            pend_o[buf] = pltpu.async_copy(
                ra.at[buf], out_h.at[pl.ds(wid * eper + b * BK, BK), :], s_o)
        for h in pend_o:
            if h is not None:
                h.wait()

    return k(tab, idx2)


def _espec(BE):
    return pl.BlockSpec((BE,), lambda i: (i,))


def _eshape(E):
    return jax.ShapeDtypeStruct((E,), jnp.float32)


def _tc_node0(posT, afT, lamS, lamE, Ws1, bs1, Ws2T, bs2, We1, be1, We2T, be2):
    def body(posT_r, afT_r, lamS_r, lamE_r, Ws1_r, bs1_r, Ws2T_r, bs2_r,
             We1_r, be1_r, We2T_r, be2_r, gbt_r, gg_r):
        p = posT_r[...]
        af = afT_r[...]
        af0 = af[0:1]
        af1 = af[1:2]
        af2 = af[2:3]
        q = af0 - 0.5
        rho = 0.1 + 0.1 * af1
        scale = 0.8 + 0.4 * af2
        or_ = rho - OFFSET
        sa = GAMMA * (rho + 0.14) ** 2
        gbt_r[...] = jnp.concatenate([p, or_, scale, q, sa], axis=0)
        gsv = jax.nn.sigmoid(
            jnp.sum(_silu(lamS_r[...] * Ws1_r[...] + bs1_r[...]) * Ws2T_r[...],
                    axis=1, keepdims=True) + bs2_r[...])
        gev = jax.nn.sigmoid(
            jnp.sum(_silu(lamE_r[...] * We1_r[...] + be1_r[...]) * We2T_r[...],
                    axis=1, keepdims=True) + be2_r[...])
        gg_r[...] = jnp.concatenate([gsv, gev], axis=1)

    return _pcall(
        body,
        out_shape=(jax.ShapeDtypeStruct((7, N), jnp.float32),
                   jax.ShapeDtypeStruct((1, 2), jnp.float32)),
    )(posT, afT, lamS, lamE, Ws1, bs1, Ws2T, bs2, We1, be1, We2T, be2)


def _tc_gb1(g1):
    def body(xs, ys, zs, orj, scj, xd, yd, zd, ori, d_r, I_r):
        ddx = xs[...] - xd[...]
        ddy = ys[...] - yd[...]
        ddz = zs[...] - zd[...]
        or_i = ori[...]
        d = jnp.sqrt(ddx * ddx + ddy * ddy + ddz * ddz + 1e-12)
        sr = scj[...] * orj[...]
        L = jnp.maximum(jnp.abs(d - sr), or_i)
        U = d + sr
        I = 0.5 * (1.0 / L - 1.0 / U
                   + 0.25 * (d - sr * sr / d) * (1.0 / (U * U) - 1.0 / (L * L))
                   + 0.5 * jnp.log(L / U) / d)
        mask = (or_i < U).astype(jnp.float32)
        d_r[...] = d
        I_r[...] = I * mask

    nb = EG // BE_G
    sp = _espec(BE_G)
    return _pcall(
        body,
        grid=(nb,),
        in_specs=[sp] * 9,
        out_specs=(sp, sp),
        out_shape=(_eshape(EG), _eshape(EG)),
    )(*g1)


def _tc_node1(Ip, gbt):
    def body(Ip_r, gbt_r, B_r, dBdI_r):
        Isum = Ip_r[0, 0:1] + Ip_r[1, 0:1]
        or_ = gbt_r[3:4]
        x = 1.0 / or_ - Isum
        xc = jnp.clip(x, 0.5, 200.0)
        B = 1.0 / xc
        mask = ((x > 0.5) & (x < 200.0)).astype(jnp.float32)
        B_r[...] = B
        dBdI_r[...] = B * B * mask

    return _pcall(
        body,
        out_shape=(jax.ShapeDtypeStruct((1, N), jnp.float32),
                   jax.ShapeDtypeStruct((1, N), jnp.float32)),
    )(Ip, gbt)


def _tc_l1(X, W1a, b1a, W1b, b1b):
    def body(*args):
        xs = args[:10]
        W1a_r, b1a_r, W1b_r, b1b_r, m_r = args[10:]
        Xb = jnp.concatenate([x[...].reshape(1, BE_N) for x in xs], axis=0)
        a1 = lax.dot_general(Xb, W1a_r[...], (((0,), (0,)), ((), ())),
                             preferred_element_type=jnp.float32) + b1a_r[...]
        m_r[...] = lax.dot_general(_silu(a1), W1b_r[...],
                                   (((1,), (0,)), ((), ())),
                                   preferred_element_type=jnp.float32) + b1b_r[...]

    nb = pl.cdiv(EN, BE_N)
    sp = _espec(BE_N)
    return _pcall(
        body,
        grid=(nb,),
        in_specs=[sp] * 10 + [
            pl.BlockSpec((10, 64), lambda i: (0, 0)),
            pl.BlockSpec((1, 64), lambda i: (0, 0)),
            pl.BlockSpec((64, 64), lambda i: (0, 0)),
            pl.BlockSpec((1, 64), lambda i: (0, 0)),
        ],
        out_specs=pl.BlockSpec((BE_N, 64), lambda i: (i, 0)),
        out_shape=jax.ShapeDtypeStruct((EN, 64), jnp.float32),
    )(*X, W1a, b1a, W1b, b1b)


def _tc_h(hp, W2as, W2ad):
    def body(hp_r, W2as_r, W2ad_r, hpre_r, Ps_r, Pd_r):
        hpre = hp_r[0] + hp_r[1]
        h = _silu(hpre)
        hpre_r[...] = hpre
        Ps_r[...] = jnp.dot(h, W2as_r[...], preferred_element_type=jnp.float32)
        Pd_r[...] = jnp.dot(h, W2ad_r[...], preferred_element_type=jnp.float32)

    nb = N // BN
    return _pcall(
        body,
        grid=(nb,),
        in_specs=[
            pl.BlockSpec((2, BN, 64), lambda i: (0, i, 0)),
            pl.BlockSpec((64, 64), lambda i: (0, 0)),
            pl.BlockSpec((64, 64), lambda i: (0, 0)),
        ],
        out_specs=(pl.BlockSpec((BN, 64), lambda i: (i, 0)),
                   pl.BlockSpec((BN, 64), lambda i: (i, 0)),
                   pl.BlockSpec((BN, 64), lambda i: (i, 0))),
        out_shape=(jax.ShapeDtypeStruct((N, 64), jnp.float32),
                   jax.ShapeDtypeStruct((N, 64), jnp.float32),
                   jax.ShapeDtypeStruct((N, 64), jnp.float32)),
    )(hp, W2as, W2ad)


def _tc_l2(Asum, b2a, W2b):
    def body(A_r, b2a_r, W2b_r, m20_r, m21_r):
        sa2 = _silu(A_r[...] + b2a_r[...])
        m2 = lax.dot_general(W2b_r[...], sa2, (((0,), (1,)), ((), ())),
                             preferred_element_type=jnp.float32)
        m20_r[...] = m2[0]
        m21_r[...] = m2[1]

    nb = pl.cdiv(EN, BE_N)
    sp = _espec(BE_N)
    return _pcall(
        body,
        grid=(nb,),
        in_specs=[
            pl.BlockSpec((BE_N, 64), lambda i: (i, 0)),
            pl.BlockSpec((1, 64), lambda i: (0, 0)),
            pl.BlockSpec((64, 2), lambda i: (0, 0)),
        ],
        out_specs=(sp, sp),
        out_shape=(_eshape(EN), _eshape(EN)),
    )(Asum, b2a, W2b)


def _tc_node2(cp, B, gbt, gg):
    def body(cp_r, B_r, gbt_r, gg_r, Bc_r, parte_r, gcp1_r, s0_r, c0_r):
        cpre0 = cp_r[0, 0:1] + cp_r[1, 0:1]
        cpre1 = cp_r[0, 1:2] + cp_r[1, 1:2]
        c0 = jax.nn.sigmoid(cpre0)
        c1 = jax.nn.sigmoid(cpre1)
        B = B_r[...]
        q = gbt_r[5:6]
        sa = gbt_r[6:7]
        gsv = gg_r[0:1, 0:1]
        gev = gg_r[0:1, 1:2]
        Bc = B * (FRACTION * c0 + (1.0 - FRACTION))
        e_self = K2 * q * q / Bc
        Bc_r[...] = Bc
        parte_r[...] = e_self * gev + sa * c1 * gsv
        gcp1_r[...] = sa * gsv * c1 * (1.0 - c1)
        s0_r[...] = c0 * (1.0 - c0)
        c0_r[...] = c0

    shp = jax.ShapeDtypeStruct((1, N), jnp.float32)
    return _pcall(
        body,
        out_shape=(shp, shp, shp, shp, shp),
    )(cp, B, gbt, gg)


def _tc_gb2(g2, d, gg):
    def body(qs, Bcs, qd, Bcd, d_r, gg_r, ep_r, gBi_r, gBj_r, gdd_r):
        q_s = qs[...]
        Bc_s = Bcs[...]
        q_d = qd[...]
        Bc_d = Bcd[...]
        d = d_r[...]
        gev = gg_r[0, 1]
        u = Bc_d * Bc_s
        ex = jnp.exp(-(d * d) / (4.0 * u))
        f2 = d * d + u * ex
        f = jnp.sqrt(f2)
        w = K2 * q_d * q_s
        e_pair = w / f
        dedf = -w / f2
        dfdd = (2.0 * d - 0.5 * d * ex) / (2.0 * f)
        dfdu = ex * (1.0 + d * d / (4.0 * u)) / (2.0 * f)
        gu = gev * dedf * dfdu
        ep_r[...] = e_pair
        gBi_r[...] = gu * Bc_s
        gBj_r[...] = gu * Bc_d
        gdd_r[...] = gev * dedf * dfdd

    nb = EG // BE_G
    sp = _espec(BE_G)
    return _pcall(
        body,
        grid=(nb,),
        in_specs=[sp] * 5 + [pl.BlockSpec((1, 2), lambda i: (0, 0))],
        out_specs=(sp, sp, sp, sp),
        out_shape=(_eshape(EG),) * 4,
    )(*g2, d, gg)


def _tc_node3(ep, parte, gbt, Bc, B, s0, c0, gcp1, gg):
    def body(ep_r, parte_r, gbt_r, Bc_r, B_r, s0_r, c0_r, gcp1_r, gg_r,
             ea_r, gcp_r, gBdir_r):
        e_gb = ep_r[0, 0:1] + ep_r[1, 0:1]
        gBi_n = ep_r[0, 1:2] + ep_r[1, 1:2]
        gBj_n = ep_r[0, 2:3] + ep_r[1, 2:3]
        q = gbt_r[5:6]
        gev = gg_r[0:1, 1:2]
        Bc = Bc_r[...]
        B = B_r[...]
        s0 = s0_r[...]
        c0 = c0_r[...]
        ea_r[...] = parte_r[...] + e_gb * gev
        gBc = gBi_n + gBj_n - gev * K2 * q * q / (Bc * Bc)
        gcp0 = gBc * B * FRACTION * s0
        gcp_r[...] = jnp.concatenate([gcp0, gcp1_r[...]], axis=0)
        gBdir_r[...] = gBc * (FRACTION * c0 + (1.0 - FRACTION))

    return _pcall(
        body,
        out_shape=(jax.ShapeDtypeStruct((1, N), jnp.float32),
                   jax.ShapeDtypeStruct((2, N), jnp.float32),
                   jax.ShapeDtypeStruct((1, N), jnp.float32)),
    )(ep, parte, gbt, Bc, B, s0, c0, gcp1, gg)


def _tc_l2b(Asum, gcpg, b2a, W2b):
    def body(A_r, g0_r, g1_r, b2a_r, W2b_r, ga2_r):
        a2 = A_r[...] + b2a_r[...]
        gm = jnp.concatenate([g0_r[...].reshape(1, BE_N),
                              g1_r[...].reshape(1, BE_N)], axis=0)
        gsa2 = lax.dot_general(gm, W2b_r[...], (((0,), (1,)), ((), ())),
                               preferred_element_type=jnp.float32)
        ga2_r[...] = gsa2 * _dsilu(a2)

    nb = pl.cdiv(EN, BE_N)
    sp = _espec(BE_N)
    return _pcall(
        body,
        grid=(nb,),
        in_specs=[
            pl.BlockSpec((BE_N, 64), lambda i: (i, 0)),
            sp, sp,
            pl.BlockSpec((1, 64), lambda i: (0, 0)),
            pl.BlockSpec((64, 2), lambda i: (0, 0)),
        ],
        out_specs=pl.BlockSpec((BE_N, 64), lambda i: (i, 0)),
        out_shape=jax.ShapeDtypeStruct((EN, 64), jnp.float32),
    )(Asum, gcpg[0], gcpg[1], b2a, W2b)


def _tc_l1bn(Gp, hpre, W2asT, W2adT, W1bT):
    def body(Gp_r, hpre_r, W2asT_r, W2adT_r, W1bT_r, R_r):
        Gs = Gp_r[0, 0] + Gp_r[1, 0]
        Gd = Gp_r[0, 1] + Gp_r[1, 1]
        gh = (jnp.dot(Gs, W2asT_r[...], preferred_element_type=jnp.float32)
              + jnp.dot(Gd, W2adT_r[...], preferred_element_type=jnp.float32))
        ghp = gh * _dsilu(hpre_r[...])
        R_r[...] = jnp.dot(ghp, W1bT_r[...], preferred_element_type=jnp.float32)

    nb = N // BN
    return _pcall(
        body,
        grid=(nb,),
        in_specs=[
            pl.BlockSpec((2, 2, BN, 64), lambda i: (0, 0, i, 0)),
            pl.BlockSpec((BN, 64), lambda i: (i, 0)),
            pl.BlockSpec((64, 64), lambda i: (0, 0)),
            pl.BlockSpec((64, 64), lambda i: (0, 0)),
            pl.BlockSpec((64, 64), lambda i: (0, 0)),
        ],
        out_specs=pl.BlockSpec((BN, 64), lambda i: (i, 0)),
        out_shape=jax.ShapeDtypeStruct((N, 64), jnp.float32),
    )(Gp, hpre, W2asT, W2adT, W1bT)


def _tc_l1b(X, Rg, W1a, b1a, Wb):
    def body(*args):
        xs = args[:10]
        Rg_r, W1a_r, b1a_r, Wb_r, gBs_r, gBd_r = args[10:]
        Xb = jnp.concatenate([x[...].reshape(1, BE_N) for x in xs], axis=0)
        a1 = lax.dot_general(Xb, W1a_r[...], (((0,), (0,)), ((), ())),
                             preferred_element_type=jnp.float32) + b1a_r[...]
        ga1 = Rg_r[...] * _dsilu(a1)
        gB = lax.dot_general(Wb_r[...], ga1, (((1,), (1,)), ((), ())),
                             preferred_element_type=jnp.float32)
        gBs_r[...] = gB[0]
        gBd_r[...] = gB[1]

    nb = pl.cdiv(EN, BE_N)
    sp = _espec(BE_N)
    return _pcall(
        body,
        grid=(nb,),
        in_specs=[sp] * 10 + [
            pl.BlockSpec((BE_N, 64), lambda i: (i, 0)),
            pl.BlockSpec((10, 64), lambda i: (0, 0)),
            pl.BlockSpec((1, 64), lambda i: (0, 0)),
            pl.BlockSpec((2, 64), lambda i: (0, 0)),
        ],
        out_specs=(sp, sp),
        out_shape=(_eshape(EN), _eshape(EN)),
    )(*X, Rg, W1a, b1a, Wb)


def _tc_node4(gp, gBdir, dBdI):
    def body(gp_r, gBdir_r, dBdI_r, gI_r):
        gB = gp_r[0, 0:1] + gp_r[1, 0:1] + gp_r[0, 1:2] + gp_r[1, 1:2]
        gI_r[...] = (gBdir_r[...] + gB) * dBdI_r[...]

    return _pcall(
        body,
        out_shape=jax.ShapeDtypeStruct((1, N), jnp.float32),
    )(gp, gBdir, dBdI)


def _tc_force(g1, d, gdd, gI):
    def body(xs, ys, zs, orj, scj, xd, yd, zd, ori, d_r, gdd_r, gI_r,
             fx_r, fy_r, fz_r):
        ddx = xs[...] - xd[...]
        ddy = ys[...] - yd[...]
        ddz = zs[...] - zd[...]
        or_i = ori[...]
        d = d_r[...]
        sr = scj[...] * orj[...]
        L = jnp.maximum(jnp.abs(d - sr), or_i)
        U = d + sr
        mask = (or_i < U).astype(jnp.float32)
        absds = jnp.abs(d - sr)
        dLdd = jnp.sign(d - sr) * (absds > or_i).astype(jnp.float32)
        iL = 1.0 / L
        iU = 1.0 / U
        idd = 1.0 / d
        t = d - sr * sr * idd
        dIdL = 0.5 * (-iL * iL + 0.5 * t * iL * iL * iL + 0.5 * iL * idd)
        dIdU = 0.5 * (iU * iU - 0.5 * t * iU * iU * iU - 0.5 * iU * idd)
        dIdd_exp = 0.5 * (0.25 * (1.0 + (sr * idd) ** 2) * (iU * iU - iL * iL)
                          - 0.5 * jnp.log(L * iU) * idd * idd)
        dIdd = (dIdL * dLdd + dIdU + dIdd_exp) * mask
        g_tot = gdd_r[...] + gI_r[...] * dIdd
        coef = g_tot * idd
        fx_r[...] = coef * ddx
        fy_r[...] = coef * ddy
        fz_r[...] = coef * ddz

    nb = EG // BE_G
    sp = _espec(BE_G)
    return _pcall(
        body,
        grid=(nb,),
        in_specs=[sp] * 12,
        out_specs=(sp, sp, sp),
        out_shape=(_eshape(EG),) * 3,
    )(*g1, d, gdd, gI)


def _tc_final(fp, e_atom, batT):
    def body(fp_r, ea_r, bat_r, F_r, en_r):
        F_r[...] = ((fp_r[0, 0:3] + fp_r[1, 0:3])
                    - (fp_r[0, 3:6] + fp_r[1, 3:6]))
        oh = (bat_r[...] == lax.broadcasted_iota(jnp.int32, (N, NB), 1)
              ).astype(jnp.float32)
        en_r[...] = lax.dot_general(ea_r[...], oh, (((1,), (0,)), ((), ())),
                                    preferred_element_type=jnp.float32)

    return _pcall(
        body,
        out_shape=(jax.ShapeDtypeStruct((3, N), jnp.float32),
                   jax.ShapeDtypeStruct((1, NB), jnp.float32)),
    )(fp, e_atom, batT)


def kernel(positions, atom_features, lambda_sterics, lambda_electrostatics,
           retrieve_forces, batch, edge_index, gnn_edge_index,
           W1a, b1a, W1b, b1b, W2a, b2a, W2b, b2b,
           Ws1, bs1, Ws2, bs2, We1, be1, We2, be2):
    posT = positions.T
    afT = atom_features.T
    src = edge_index[0].astype(jnp.int32)
    dst = edge_index[1].astype(jnp.int32)
    gs = gnn_edge_index[0].astype(jnp.int32)
    gd = gnn_edge_index[1].astype(jnp.int32)
    src2 = src.reshape(EG // C, C)
    dst2 = dst.reshape(EG // C, C)
    gs2 = gs.reshape(EN // C, C)
    gd2 = gd.reshape(EN // C, C)
    batT = batch.astype(jnp.int32).reshape(N, 1)

    gbt, gg = _tc_node0(
        posT, afT,
        lambda_sterics.reshape(1, 1), lambda_electrostatics.reshape(1, 1),
        Ws1, bs1.reshape(1, 32), Ws2.reshape(1, 32), bs2.reshape(1, 1),
        We1, be1.reshape(1, 32), We2.reshape(1, 32), be2.reshape(1, 1))

    g1 = _sc_gather_scalars(gbt.reshape(7 * N), src, (0, 1, 2, 3, 4),
                            dst, (0, 1, 2, 3), EG)
    d, I = _tc_gb1(g1)
    IpF = _sc_scatter_scalars([I], [dst2], [(0, 0, 0)], EG, 1)
    Ip = IpF.reshape(2, 1, NP)[:, :, :N]
    B, dBdI = _tc_node1(Ip, gbt)

    gnt = jnp.concatenate([B, afT[:4]], axis=0)
    X = _sc_gather_scalars(gnt.reshape(5 * N), gs, (0, 1, 2, 3, 4),
                           gd, (0, 1, 2, 3, 4), EN)
    m = _tc_l1(X, W1a, b1a.reshape(1, 64), W1b, b1b.reshape(1, 64))
    hp = _sc_scatter_rows(m, gd2, None)
    hpre, Ps, Pd = _tc_h(hp[:, 0], W2a[:64], W2a[64:])

    Asum = _sc_gather_rows_sum2(Ps, Pd, gs2, gd2)
    m2 = _tc_l2(Asum, b2a.reshape(1, 64), W2b)
    cpF = _sc_scatter_scalars(list(m2), [gd2], [(0, 0, 0), (1, 0, 1)], EN, 2)
    cp = cpF.reshape(2, 2, NP)[:, :, :N]
    Bc, parte, gcp1, s0, c0 = _tc_node2(cp, B, gbt, gg)

    qBc = jnp.concatenate([gbt[5:6], Bc], axis=0)
    g2 = _sc_gather_scalars(qBc.reshape(2 * N), src, (0, 1), dst, (0, 1), EG)
    e_pair, gBi, gBj, gdd = _tc_gb2(g2, d, gg)
    epF = _sc_scatter_scalars([e_pair, gBi, gBj], [dst2, src2],
                              [(0, 0, 0), (1, 0, 1), (2, 1, 2)], EG, 3)
    ep = epF.reshape(2, 3, NP)[:, :, :N]
    e_atom, gcp, gBdir = _tc_node3(ep, parte, gbt, Bc, B, s0, c0, gcp1, gg)

    gcpg = _sc_gather_scalars(gcp.reshape(2 * N), gd, (0, 1), None, None, EN)
    ga2 = _tc_l2b(Asum, gcpg, b2a.reshape(1, 64), W2b)
    Gp = _sc_scatter_rows(ga2, gs2, gd2)
    Rm = _tc_l1bn(Gp, hpre, W2a[:64].T, W2a[64:].T, W1b.T)
    Rg = _sc_gather_rows(Rm, gd2)
    Wb = jnp.concatenate([W1a[0:1], W1a[5:6]], axis=0)
    gB2 = _tc_l1b(X, Rg, W1a, b1a.reshape(1, 64), Wb)
    gpF = _sc_scatter_scalars(list(gB2), [gs2, gd2],
                              [(0, 0, 0), (1, 1, 1)], EN, 2)
    gp = gpF.reshape(2, 2, NP)[:, :, :N]
    gIsum = _tc_node4(gp, gBdir, dBdI)

    (gI,) = _sc_gather_scalars(gIsum.reshape(N), dst, (0,), None, None, EG)
    fv = _tc_force(g1, d, gdd, gI)
    fpF = _sc_scatter_scalars(
        list(fv), [dst2, src2],
        [(0, 0, 0), (1, 0, 1), (2, 0, 2), (0, 1, 3), (1, 1, 4), (2, 1, 5)],
        EG, 6)
    fp = fpF.reshape(2, 6, NP)[:, :, :N]
    F, en = _tc_final(fp, e_atom, batT)

    energy = en.reshape(NB, 1)
    forces = F.T
    return energy, forces


# speedup vs baseline: 44.4315x; 1.0993x over previous
import functools

import jax
import jax.numpy as jnp
from jax import lax
from jax.experimental import pallas as pl
from jax.experimental.pallas import tpu as pltpu
from jax.experimental.pallas import tpu_sc as plsc

N = 10000
EG = 640000
EN = 320000
NB = 64
FRACTION = 0.5
GAMMA = 0.00542
OFFSET = 0.0195141
COUL = 138.935485
EPS_FAC = 1.0 - 1.0 / 78.5
K2 = -0.5 * COUL * EPS_FAC

NP = 10240
CS = 2000
C = 80
BE_G = 5120
BE_N = 8192
BN = 2000
NC = 2
NS = 16
NW = NC * NS


def _silu(x):
    return x * jax.nn.sigmoid(x)


def _dsilu(x):
    s = jax.nn.sigmoid(x)
    return s * (1.0 + x * (1.0 - s))


def _pcall(body, **kw):
    return pl.pallas_call(body, **kw)


_SC_PARAMS = dict(
    compiler_params=pltpu.CompilerParams(use_tc_tiling_on_sc=False,
                                         needs_layout_passes=False))


def _sc_mesh():
    return plsc.VectorSubcoreMesh(core_axis_name="c", subcore_axis_name="s")


def _wid():
    return lax.axis_index("s") * NC + lax.axis_index("c")


def _sc_gather_scalars(tbl, idxA, rowsA, idxB, rowsB, E):
    T = tbl.shape[0] // N
    R = len(rowsA) + (len(rowsB) if rowsB is not None else 0)
    eper = E // NW
    nch = eper // CS

    @functools.partial(
        pl.kernel,
        mesh=_sc_mesh(),
        out_type=tuple(jax.ShapeDtypeStruct((E,), jnp.float32)
                       for _ in range(R)),
        scratch_types=[
            pltpu.VMEM((T * N,), jnp.float32),
            pltpu.VMEM((2, CS), jnp.int32),
            pltpu.VMEM((2, CS), jnp.int32),
            pltpu.VMEM((2, R, CS), jnp.float32),
            pltpu.SemaphoreType.DMA,
            pltpu.SemaphoreType.DMA,
            pltpu.SemaphoreType.DMA,
        ],
        **_SC_PARAMS,
    )
    def k(tbl_h, idxA_h, idxB_h, *rest):
        out_hs = rest[:R]
        tbl_v, ia_v, ib_v, out_v, s_t, s_l, s_o = rest[R:]
        wid = _wid()
        tcp = pltpu.async_copy(tbl_h, tbl_v, s_t)

        def load(b, buf):
            base = wid * eper + b * CS
            h = [pltpu.async_copy(idxA_h.at[pl.ds(base, CS)], ia_v.at[buf],
                                  s_l)]
            if rowsB is not None:
                h.append(pltpu.async_copy(idxB_h.at[pl.ds(base, CS)],
                                          ib_v.at[buf], s_l))
            return h

        pend_l = load(0, 0)
        tcp.wait()
        pend_o = [[], []]
        for b in range(nch):
            buf = b & 1
            for h in pend_l:
                h.wait()
            if b + 1 < nch:
                pend_l = load(b + 1, buf ^ 1)
            for h in pend_o[buf]:
                h.wait()

            def inner(i, carry, _buf=buf):
                for u in range(5):
                    off = i * 80 + u * 16
                    iva = ia_v[_buf, pl.ds(off, 16)]
                    r = 0
                    for row in rowsA:
                        v = plsc.load_gather(tbl_v, [iva + row * N])
                        out_v[_buf, r, pl.ds(off, 16)] = v
                        r += 1
                    if rowsB is not None:
                        ivb = ib_v[_buf, pl.ds(off, 16)]
                        for row in rowsB:
                            v = plsc.load_gather(tbl_v, [ivb + row * N])
                            out_v[_buf, r, pl.ds(off, 16)] = v
                            r += 1
                return carry

            lax.fori_loop(0, CS // 80, inner, 0)
            base = wid * eper + b * CS
            pend_o[buf] = [
                pltpu.async_copy(out_v.at[buf, r],
                                 out_hs[r].at[pl.ds(base, CS)], s_o)
                for r in range(R)]
        for hb in pend_o:
            for h in hb:
                h.wait()

    idxB_arg = idxB if idxB is not None else idxA
    return k(tbl, idxA, idxB_arg)


def _sc_scatter_scalars(vals, idxs, specs, E, nslots):
    eper = E // NW
    BK = 2000 if eper % 2000 == 0 else 400
    W = BK // C
    nblk = eper // BK
    ACC = nslots * NP
    stripe = ACC // NS
    nv = len(vals)
    ni = len(idxs)
    ns = len(specs)

    @functools.partial(
        pl.kernel,
        mesh=_sc_mesh(),
        out_type=jax.ShapeDtypeStruct((2, ACC), jnp.float32),
        scratch_types=[
            pltpu.VMEM_SHARED((ACC,), jnp.float32),
            pltpu.VMEM((stripe,), jnp.float32),
            pltpu.VMEM((ni, W, C), jnp.int32),
            pltpu.VMEM((ns, W, C), jnp.int32),
            pltpu.VMEM((ns, BK), jnp.float32),
            pltpu.SemaphoreType.DMA,
            pltpu.SemaphoreType.DMA,
        ],
        **_SC_PARAMS,
    )
    def k(*args):
        vals_hs = args[:nv]
        idx_hs = args[nv:nv + ni]
        out_h = args[nv + ni]
        acc, zb, iv, io, vv, s_l, s_s = args[nv + ni + 1:]
        cc = lax.axis_index("c")
        sid = lax.axis_index("s")
        wid = _wid()

        def zloop(i, carry):
            zb[pl.ds(i * 16, 16)] = jnp.zeros((16,), jnp.float32)
            return carry

        lax.fori_loop(0, stripe // 16, zloop, 0)
        pltpu.sync_copy(zb, acc.at[pl.ds(sid * stripe, stripe)])
        plsc.subcore_barrier()

        def blk(b, carry):
            irow = wid * (eper // C) + b * W
            hs = []
            for t in range(ni):
                hs.append(pltpu.async_copy(idx_hs[t].at[pl.ds(irow, W)],
                                           iv.at[t], s_l))
            for si, (vr, ii, slot) in enumerate(specs):
                hs.append(pltpu.async_copy(
                    vals_hs[vr].at[pl.ds(wid * eper + b * BK, BK)],
                    vv.at[si], s_l))
            for h in hs:
                h.wait()
            for si, (vr, ii, slot) in enumerate(specs):
                def offl(i, carry2, _si=si, _ii=ii, _slot=slot):
                    for w in range(W):
                        io[_si, w, pl.ds(i * 16, 16)] = (
                            iv[_ii, w, pl.ds(i * 16, 16)] + _slot * NP)
                    return carry2

                lax.fori_loop(0, C // 16, offl, 0)
            ss = []
            for si in range(ns):
                for w in range(W):
                    ss.append(pltpu.async_copy(
                        vv.at[si, pl.ds(w * C, C)], acc.at[io.at[si, w]],
                        s_s, add=True))
            for h in ss:
                h.wait()
            return carry

        lax.fori_loop(0, nblk, blk, 0)
        plsc.subcore_barrier()
        pltpu.sync_copy(acc.at[pl.ds(sid * stripe, stripe)],
                        out_h.at[cc, pl.ds(sid * stripe, stripe)])

    return k(*vals, *idxs)


def _sc_scatter_rows(vals, idxA2, idxB2):
    eper = EN // NW
    BK = 400
    W = BK // C
    nblk = eper // BK
    dual = idxB2 is not None
    nsets = 2 if dual else 1
    rstripe = N // NS

    scratch = [
        pltpu.VMEM_SHARED((N, 64), jnp.float32),
        pltpu.VMEM((rstripe // 5, 64), jnp.float32),
        pltpu.VMEM((nsets, W, C), jnp.int32),
        pltpu.VMEM((BK, 64), jnp.float32),
        pltpu.SemaphoreType.DMA,
        pltpu.SemaphoreType.DMA,
    ]
    if dual:
        scratch.insert(1, pltpu.VMEM_SHARED((N, 64), jnp.float32))

    @functools.partial(
        pl.kernel,
        mesh=_sc_mesh(),
        out_type=jax.ShapeDtypeStruct((2, nsets, N, 64), jnp.float32),
        scratch_types=scratch,
        **_SC_PARAMS,
    )
    def k(vals_h, idxA_h, idxB_h, out_h, *scr):
        if dual:
            accA, accB, zb, iv, vv, s_l, s_s = scr
            accs = (accA, accB)
            idx_hs = (idxA_h, idxB_h)
        else:
            accA, zb, iv, vv, s_l, s_s = scr
            accs = (accA,)
            idx_hs = (idxA_h,)
        cc = lax.axis_index("c")
        sid = lax.axis_index("s")
        wid = _wid()

        def zloop(i, carry):
            for c4 in range(4):
                zb[i, pl.ds(c4 * 16, 16)] = jnp.zeros((16,), jnp.float32)
            return carry

        lax.fori_loop(0, rstripe // 5, zloop, 0)
        for a in accs:
            for p in range(5):
                pltpu.sync_copy(
                    zb, a.at[pl.ds(sid * rstripe + p * (rstripe // 5),
                                   rstripe // 5), :])
        plsc.subcore_barrier()

        def blk(b, carry):
            irow = wid * (eper // C) + b * W
            hs = [pltpu.async_copy(vals_h.at[pl.ds(wid * eper + b * BK, BK), :],
                                   vv, s_l)]
            for t in range(nsets):
                hs.append(pltpu.async_copy(idx_hs[t].at[pl.ds(irow, W)],
                                           iv.at[t], s_l))
            for h in hs:
                h.wait()
            ss = []
            for t in range(nsets):
                for w in range(W):
                    ss.append(pltpu.async_copy(
                        vv.at[pl.ds(w * C, C), :], accs[t].at[iv.at[t, w]],
                        s_s, add=True))
            for h in ss:
                h.wait()
            return carry

        lax.fori_loop(0, nblk, blk, 0)
        plsc.subcore_barrier()
        for t in range(nsets):
            pltpu.sync_copy(accs[t].at[pl.ds(sid * rstripe, rstripe), :],
                            out_h.at[cc, t, pl.ds(sid * rstripe, rstripe), :])

    idxB_arg = idxB2 if dual else idxA2
    return k(vals, idxA2, idxB_arg)


def _sc_gather_rows_sum2(Ps, Pd, gs2, gd2):
    eper = EN // NW
    BK = 400
    W = BK // C
    nblk = eper // BK

    @functools.partial(
        pl.kernel,
        mesh=_sc_mesh(),
        out_type=jax.ShapeDtypeStruct((EN, 64), jnp.float32),
        scratch_types=[
            pltpu.VMEM((2, W, C), jnp.int32),
            pltpu.VMEM((2, BK, 64), jnp.float32),
            pltpu.VMEM((2, BK, 64), jnp.float32),
            pltpu.SemaphoreType.DMA,
            pltpu.SemaphoreType.DMA,
            pltpu.SemaphoreType.DMA,
        ],
        **_SC_PARAMS,
    )
    def k(Ps_h, Pd_h, gs_h, gd_h, out_h, iv, ra, rb, s_l, s_g, s_o):
        wid = _wid()
        pend_o = [None, None]
        for b in range(nblk):
            buf = b & 1
            irow = wid * (eper // C) + b * W
            h1 = pltpu.async_copy(gs_h.at[pl.ds(irow, W)], iv.at[0], s_l)
            h2 = pltpu.async_copy(gd_h.at[pl.ds(irow, W)], iv.at[1], s_l)
            h1.wait()
            h2.wait()
            if pend_o[buf] is not None:
                pend_o[buf].wait()
            gh = []
            for w in range(W):
                gh.append(pltpu.async_copy(
                    Ps_h.at[iv.at[0, w]],
                    ra.at[buf, pl.ds(w * C, C), :], s_g))
                gh.append(pltpu.async_copy(
                    Pd_h.at[iv.at[1, w]],
                    rb.at[buf, pl.ds(w * C, C), :], s_g))
            for h in gh:
                h.wait()

            def addl(i, carry2, _buf=buf):
                for c4 in range(4):
                    ra[_buf, i, pl.ds(c4 * 16, 16)] = (
                        ra[_buf, i, pl.ds(c4 * 16, 16)]
                        + rb[_buf, i, pl.ds(c4 * 16, 16)])
                return carry2

            lax.fori_loop(0, BK, addl, 0)
            pend_o[buf] = pltpu.async_copy(
                ra.at[buf], out_h.at[pl.ds(wid * eper + b * BK, BK), :], s_o)
        for h in pend_o:
            if h is not None:
                h.wait()

    return k(Ps, Pd, gs2, gd2)


def _sc_gather_rows(tab, idx2):
    eper = EN // NW
    BK = 400
    W = BK // C
    nblk = eper // BK

    @functools.partial(
        pl.kernel,
        mesh=_sc_mesh(),
        out_type=jax.ShapeDtypeStruct((EN, 64), jnp.float32),
        scratch_types=[
            pltpu.VMEM((W, C), jnp.int32),
            pltpu.VMEM((2, BK, 64), jnp.float32),
            pltpu.SemaphoreType.DMA,
            pltpu.SemaphoreType.DMA,
            pltpu.SemaphoreType.DMA,
        ],
        **_SC_PARAMS,
    )
    def k(tab_h, idx_h, out_h, iv, ra, s_l, s_g, s_o):
        wid = _wid()
        pend_o = [None, None]
        for b in range(nblk):
            buf = b & 1
            irow = wid * (eper // C) + b * W
            pltpu.async_copy(idx_h.at[pl.ds(irow, W)], iv, s_l).wait()
            if pend_o[buf] is not None:
                pend_o[buf].wait()
            gh = []
            for w in range(W):
                gh.append(pltpu.async_copy(
                    tab_h.at[iv.at[w]], ra.at[buf, pl.ds(w * C, C), :], s_g))
            for h in gh:
                h.wait()
            pend_o[buf] = pltpu.async_copy(
                ra.at[buf], out_h.at[pl.ds(wid * eper + b * BK, BK), :], s_o)
        for h in pend_o:
            if h is not None:
                h.wait()

    return k(tab, idx2)


def _espec(BE):
    return pl.BlockSpec((BE,), lambda i: (i,))


def _eshape(E):
    return jax.ShapeDtypeStruct((E,), jnp.float32)


def _tc_node0(posT, afT, lamS, lamE, Ws1, bs1, Ws2T, bs2, We1, be1, We2T, be2):
    def body(posT_r, afT_r, lamS_r, lamE_r, Ws1_r, bs1_r, Ws2T_r, bs2_r,
             We1_r, be1_r, We2T_r, be2_r, gbt_r, gg_r):
        p = posT_r[...]
        af = afT_r[...]
        af0 = af[0:1]
        af1 = af[1:2]
        af2 = af[2:3]
        q = af0 - 0.5
        rho = 0.1 + 0.1 * af1
        scale = 0.8 + 0.4 * af2
        or_ = rho - OFFSET
        sa = GAMMA * (rho + 0.14) ** 2
        gbt_r[...] = jnp.concatenate([p, or_, scale, q, sa], axis=0)
        gsv = jax.nn.sigmoid(
            jnp.sum(_silu(lamS_r[...] * Ws1_r[...] + bs1_r[...]) * Ws2T_r[...],
                    axis=1, keepdims=True) + bs2_r[...])
        gev = jax.nn.sigmoid(
            jnp.sum(_silu(lamE_r[...] * We1_r[...] + be1_r[...]) * We2T_r[...],
                    axis=1, keepdims=True) + be2_r[...])
        gg_r[...] = jnp.concatenate([gsv, gev], axis=1)

    return _pcall(
        body,
        out_shape=(jax.ShapeDtypeStruct((7, N), jnp.float32),
                   jax.ShapeDtypeStruct((1, 2), jnp.float32)),
    )(posT, afT, lamS, lamE, Ws1, bs1, Ws2T, bs2, We1, be1, We2T, be2)


def _tc_gb1(g1):
    def body(xs, ys, zs, orj, scj, xd, yd, zd, ori, d_r, I_r):
        ddx = xs[...] - xd[...]
        ddy = ys[...] - yd[...]
        ddz = zs[...] - zd[...]
        or_i = ori[...]
        d = jnp.sqrt(ddx * ddx + ddy * ddy + ddz * ddz + 1e-12)
        sr = scj[...] * orj[...]
        L = jnp.maximum(jnp.abs(d - sr), or_i)
        U = d + sr
        I = 0.5 * (1.0 / L - 1.0 / U
                   + 0.25 * (d - sr * sr / d) * (1.0 / (U * U) - 1.0 / (L * L))
                   + 0.5 * jnp.log(L / U) / d)
        mask = (or_i < U).astype(jnp.float32)
        d_r[...] = d
        I_r[...] = I * mask

    nb = EG // BE_G
    sp = _espec(BE_G)
    return _pcall(
        body,
        grid=(nb,),
        in_specs=[sp] * 9,
        out_specs=(sp, sp),
        out_shape=(_eshape(EG), _eshape(EG)),
    )(*g1)


def _tc_node1(Ip, gbt):
    def body(Ip_r, gbt_r, B_r, dBdI_r):
        Isum = Ip_r[0, 0:1] + Ip_r[1, 0:1]
        or_ = gbt_r[3:4]
        x = 1.0 / or_ - Isum
        xc = jnp.clip(x, 0.5, 200.0)
        B = 1.0 / xc
        mask = ((x > 0.5) & (x < 200.0)).astype(jnp.float32)
        B_r[...] = B
        dBdI_r[...] = B * B * mask

    return _pcall(
        body,
        out_shape=(jax.ShapeDtypeStruct((1, N), jnp.float32),
                   jax.ShapeDtypeStruct((1, N), jnp.float32)),
    )(Ip, gbt)


def _tc_l1(X, W1a, b1a, W1b, b1b):
    def body(*args):
        xs = args[:10]
        W1a_r, b1a_r, W1b_r, b1b_r, m_r = args[10:]
        Xb = jnp.concatenate([x[...].reshape(1, BE_N) for x in xs], axis=0)
        a1 = lax.dot_general(Xb, W1a_r[...], (((0,), (0,)), ((), ())),
                             preferred_element_type=jnp.float32) + b1a_r[...]
        m_r[...] = lax.dot_general(_silu(a1), W1b_r[...],
                                   (((1,), (0,)), ((), ())),
                                   preferred_element_type=jnp.float32) + b1b_r[...]

    nb = pl.cdiv(EN, BE_N)
    sp = _espec(BE_N)
    return _pcall(
        body,
        grid=(nb,),
        in_specs=[sp] * 10 + [
            pl.BlockSpec((10, 64), lambda i: (0, 0)),
            pl.BlockSpec((1, 64), lambda i: (0, 0)),
            pl.BlockSpec((64, 64), lambda i: (0, 0)),
            pl.BlockSpec((1, 64), lambda i: (0, 0)),
        ],
        out_specs=pl.BlockSpec((BE_N, 64), lambda i: (i, 0)),
        out_shape=jax.ShapeDtypeStruct((EN, 64), jnp.float32),
    )(*X, W1a, b1a, W1b, b1b)


def _tc_h(hp, W2as, W2ad):
    def body(hp_r, W2as_r, W2ad_r, hpre_r, Ps_r, Pd_r):
        hpre = hp_r[0] + hp_r[1]
        h = _silu(hpre)
        hpre_r[...] = hpre
        Ps_r[...] = jnp.dot(h, W2as_r[...], preferred_element_type=jnp.float32)
        Pd_r[...] = jnp.dot(h, W2ad_r[...], preferred_element_type=jnp.float32)

    nb = N // BN
    return _pcall(
        body,
        grid=(nb,),
        in_specs=[
            pl.BlockSpec((2, BN, 64), lambda i: (0, i, 0)),
            pl.BlockSpec((64, 64), lambda i: (0, 0)),
            pl.BlockSpec((64, 64), lambda i: (0, 0)),
        ],
        out_specs=(pl.BlockSpec((BN, 64), lambda i: (i, 0)),
                   pl.BlockSpec((BN, 64), lambda i: (i, 0)),
                   pl.BlockSpec((BN, 64), lambda i: (i, 0))),
        out_shape=(jax.ShapeDtypeStruct((N, 64), jnp.float32),
                   jax.ShapeDtypeStruct((N, 64), jnp.float32),
                   jax.ShapeDtypeStruct((N, 64), jnp.float32)),
    )(hp, W2as, W2ad)


def _tc_l2(Asum, b2a, W2b):
    def body(A_r, b2a_r, W2b_r, m20_r, m21_r):
        sa2 = _silu(A_r[...] + b2a_r[...])
        m2 = lax.dot_general(W2b_r[...], sa2, (((0,), (1,)), ((), ())),
                             preferred_element_type=jnp.float32)
        m20_r[...] = m2[0]
        m21_r[...] = m2[1]

    nb = pl.cdiv(EN, BE_N)
    sp = _espec(BE_N)
    return _pcall(
        body,
        grid=(nb,),
        in_specs=[
            pl.BlockSpec((BE_N, 64), lambda i: (i, 0)),
            pl.BlockSpec((1, 64), lambda i: (0, 0)),
            pl.BlockSpec((64, 2), lambda i: (0, 0)),
        ],
        out_specs=(sp, sp),
        out_shape=(_eshape(EN), _eshape(EN)),
    )(Asum, b2a, W2b)


def _tc_node2(cp, B, gbt, gg):
    def body(cp_r, B_r, gbt_r, gg_r, Bc_r, parte_r, gcp1_r, s0_r, c0_r):
        cpre0 = cp_r[0, 0:1] + cp_r[1, 0:1]
        cpre1 = cp_r[0, 1:2] + cp_r[1, 1:2]
        c0 = jax.nn.sigmoid(cpre0)
        c1 = jax.nn.sigmoid(cpre1)
        B = B_r[...]
        q = gbt_r[5:6]
        sa = gbt_r[6:7]
        gsv = gg_r[0:1, 0:1]
        gev = gg_r[0:1, 1:2]
        Bc = B * (FRACTION * c0 + (1.0 - FRACTION))
        e_self = K2 * q * q / Bc
        Bc_r[...] = Bc
        parte_r[...] = e_self * gev + sa * c1 * gsv
        gcp1_r[...] = sa * gsv * c1 * (1.0 - c1)
        s0_r[...] = c0 * (1.0 - c0)
        c0_r[...] = c0

    shp = jax.ShapeDtypeStruct((1, N), jnp.float32)
    return _pcall(
        body,
        out_shape=(shp, shp, shp, shp, shp),
    )(cp, B, gbt, gg)


def _tc_gb2(g2, d, gg):
    def body(qs, Bcs, qd, Bcd, d_r, gg_r, ep_r, gBi_r, gBj_r, gdd_r):
        q_s = qs[...]
        Bc_s = Bcs[...]
        q_d = qd[...]
        Bc_d = Bcd[...]
        d = d_r[...]
        gev = gg_r[0, 1]
        u = Bc_d * Bc_s
        ex = jnp.exp(-(d * d) / (4.0 * u))
        f2 = d * d + u * ex
        f = jnp.sqrt(f2)
        w = K2 * q_d * q_s
        e_pair = w / f
        dedf = -w / f2
        dfdd = (2.0 * d - 0.5 * d * ex) / (2.0 * f)
        dfdu = ex * (1.0 + d * d / (4.0 * u)) / (2.0 * f)
        gu = gev * dedf * dfdu
        ep_r[...] = e_pair
        gBi_r[...] = gu * Bc_s
        gBj_r[...] = gu * Bc_d
        gdd_r[...] = gev * dedf * dfdd

    nb = EG // BE_G
    sp = _espec(BE_G)
    return _pcall(
        body,
        grid=(nb,),
        in_specs=[sp] * 5 + [pl.BlockSpec((1, 2), lambda i: (0, 0))],
        out_specs=(sp, sp, sp, sp),
        out_shape=(_eshape(EG),) * 4,
    )(*g2, d, gg)


def _tc_node3(ep, parte, gbt, Bc, B, s0, c0, gcp1, gg):
    def body(ep_r, parte_r, gbt_r, Bc_r, B_r, s0_r, c0_r, gcp1_r, gg_r,
             ea_r, gcp_r, gBdir_r):
        e_gb = ep_r[0, 0:1] + ep_r[1, 0:1]
        gBi_n = ep_r[0, 1:2] + ep_r[1, 1:2]
        gBj_n = ep_r[0, 2:3] + ep_r[1, 2:3]
        q = gbt_r[5:6]
        gev = gg_r[0:1, 1:2]
        Bc = Bc_r[...]
        B = B_r[...]
        s0 = s0_r[...]
        c0 = c0_r[...]
        ea_r[...] = parte_r[...] + e_gb * gev
        gBc = gBi_n + gBj_n - gev * K2 * q * q / (Bc * Bc)
        gcp0 = gBc * B * FRACTION * s0
        gcp_r[...] = jnp.concatenate([gcp0, gcp1_r[...]], axis=0)
        gBdir_r[...] = gBc * (FRACTION * c0 + (1.0 - FRACTION))

    return _pcall(
        body,
        out_shape=(jax.ShapeDtypeStruct((1, N), jnp.float32),
                   jax.ShapeDtypeStruct((2, N), jnp.float32),
                   jax.ShapeDtypeStruct((1, N), jnp.float32)),
    )(ep, parte, gbt, Bc, B, s0, c0, gcp1, gg)


def _tc_l2b(Asum, gcpg, b2a, W2b):
    def body(A_r, g0_r, g1_r, b2a_r, W2b_r, ga2_r):
        a2 = A_r[...] + b2a_r[...]
        gm = jnp.concatenate([g0_r[...].reshape(1, BE_N),
                              g1_r[...].reshape(1, BE_N)], axis=0)
        gsa2 = lax.dot_general(gm, W2b_r[...], (((0,), (1,)), ((), ())),
                               preferred_element_type=jnp.float32)
        ga2_r[...] = gsa2 * _dsilu(a2)

    nb = pl.cdiv(EN, BE_N)
    sp = _espec(BE_N)
    return _pcall(
        body,
        grid=(nb,),
        in_specs=[
            pl.BlockSpec((BE_N, 64), lambda i: (i, 0)),
            sp, sp,
            pl.BlockSpec((1, 64), lambda i: (0, 0)),
            pl.BlockSpec((64, 2), lambda i: (0, 0)),
        ],
        out_specs=pl.BlockSpec((BE_N, 64), lambda i: (i, 0)),
        out_shape=jax.ShapeDtypeStruct((EN, 64), jnp.float32),
    )(Asum, gcpg[0], gcpg[1], b2a, W2b)


def _tc_l1bn(Gp, hpre, W2asT, W2adT, W1bT):
    def body(Gp_r, hpre_r, W2asT_r, W2adT_r, W1bT_r, R_r):
        Gs = Gp_r[0, 0] + Gp_r[1, 0]
        Gd = Gp_r[0, 1] + Gp_r[1, 1]
        gh = (jnp.dot(Gs, W2asT_r[...], preferred_element_type=jnp.float32)
              + jnp.dot(Gd, W2adT_r[...], preferred_element_type=jnp.float32))
        ghp = gh * _dsilu(hpre_r[...])
        R_r[...] = jnp.dot(ghp, W1bT_r[...], preferred_element_type=jnp.float32)

    nb = N // BN
    return _pcall(
        body,
        grid=(nb,),
        in_specs=[
            pl.BlockSpec((2, 2, BN, 64), lambda i: (0, 0, i, 0)),
            pl.BlockSpec((BN, 64), lambda i: (i, 0)),
            pl.BlockSpec((64, 64), lambda i: (0, 0)),
            pl.BlockSpec((64, 64), lambda i: (0, 0)),
            pl.BlockSpec((64, 64), lambda i: (0, 0)),
        ],
        out_specs=pl.BlockSpec((BN, 64), lambda i: (i, 0)),
        out_shape=jax.ShapeDtypeStruct((N, 64), jnp.float32),
    )(Gp, hpre, W2asT, W2adT, W1bT)


def _tc_l1b(X, Rg, W1a, b1a, Wb):
    def body(*args):
        xs = args[:10]
        Rg_r, W1a_r, b1a_r, Wb_r, gBs_r, gBd_r = args[10:]
        Xb = jnp.concatenate([x[...].reshape(1, BE_N) for x in xs], axis=0)
        a1 = lax.dot_general(Xb, W1a_r[...], (((0,), (0,)), ((), ())),
                             preferred_element_type=jnp.float32) + b1a_r[...]
        ga1 = Rg_r[...] * _dsilu(a1)
        gB = lax.dot_general(Wb_r[...], ga1, (((1,), (1,)), ((), ())),
                             preferred_element_type=jnp.float32)
        gBs_r[...] = gB[0]
        gBd_r[...] = gB[1]

    nb = pl.cdiv(EN, BE_N)
    sp = _espec(BE_N)
    return _pcall(
        body,
        grid=(nb,),
        in_specs=[sp] * 10 + [
            pl.BlockSpec((BE_N, 64), lambda i: (i, 0)),
            pl.BlockSpec((10, 64), lambda i: (0, 0)),
            pl.BlockSpec((1, 64), lambda i: (0, 0)),
            pl.BlockSpec((2, 64), lambda i: (0, 0)),
        ],
        out_specs=(sp, sp),
        out_shape=(_eshape(EN), _eshape(EN)),
    )(*X, Rg, W1a, b1a, Wb)


def _tc_node4(gp, gBdir, dBdI):
    def body(gp_r, gBdir_r, dBdI_r, gI_r):
        gB = gp_r[0, 0:1] + gp_r[1, 0:1] + gp_r[0, 1:2] + gp_r[1, 1:2]
        gI_r[...] = (gBdir_r[...] + gB) * dBdI_r[...]

    return _pcall(
        body,
        out_shape=jax.ShapeDtypeStruct((1, N), jnp.float32),
    )(gp, gBdir, dBdI)


def _tc_force(g1, d, gdd, gI):
    def body(xs, ys, zs, orj, scj, xd, yd, zd, ori, d_r, gdd_r, gI_r,
             fx_r, fy_r, fz_r):
        ddx = xs[...] - xd[...]
        ddy = ys[...] - yd[...]
        ddz = zs[...] - zd[...]
        or_i = ori[...]
        d = d_r[...]
        sr = scj[...] * orj[...]
        L = jnp.maximum(jnp.abs(d - sr), or_i)
        U = d + sr
        mask = (or_i < U).astype(jnp.float32)
        absds = jnp.abs(d - sr)
        dLdd = jnp.sign(d - sr) * (absds > or_i).astype(jnp.float32)
        iL = 1.0 / L
        iU = 1.0 / U
        idd = 1.0 / d
        t = d - sr * sr * idd
        dIdL = 0.5 * (-iL * iL + 0.5 * t * iL * iL * iL + 0.5 * iL * idd)
        dIdU = 0.5 * (iU * iU - 0.5 * t * iU * iU * iU - 0.5 * iU * idd)
        dIdd_exp = 0.5 * (0.25 * (1.0 + (sr * idd) ** 2) * (iU * iU - iL * iL)
                          - 0.5 * jnp.log(L * iU) * idd * idd)
        dIdd = (dIdL * dLdd + dIdU + dIdd_exp) * mask
        g_tot = gdd_r[...] + gI_r[...] * dIdd
        coef = g_tot * idd
        fx_r[...] = coef * ddx
        fy_r[...] = coef * ddy
        fz_r[...] = coef * ddz

    nb = EG // BE_G
    sp = _espec(BE_G)
    return _pcall(
        body,
        grid=(nb,),
        in_specs=[sp] * 12,
        out_specs=(sp, sp, sp),
        out_shape=(_eshape(EG),) * 3,
    )(*g1, d, gdd, gI)


def _tc_final(fp, e_atom, batT):
    def body(fp_r, ea_r, bat_r, F_r, en_r):
        F_r[...] = ((fp_r[0, 0:3] + fp_r[1, 0:3])
                    - (fp_r[0, 3:6] + fp_r[1, 3:6]))
        oh = (bat_r[...] == lax.broadcasted_iota(jnp.int32, (N, NB), 1)
              ).astype(jnp.float32)
        en_r[...] = lax.dot_general(ea_r[...], oh, (((1,), (0,)), ((), ())),
                                    preferred_element_type=jnp.float32)

    return _pcall(
        body,
        out_shape=(jax.ShapeDtypeStruct((3, N), jnp.float32),
                   jax.ShapeDtypeStruct((1, NB), jnp.float32)),
    )(fp, e_atom, batT)


def kernel(positions, atom_features, lambda_sterics, lambda_electrostatics,
           retrieve_forces, batch, edge_index, gnn_edge_index,
           W1a, b1a, W1b, b1b, W2a, b2a, W2b, b2b,
           Ws1, bs1, Ws2, bs2, We1, be1, We2, be2):
    posT = positions.T
    afT = atom_features.T
    src = edge_index[0].astype(jnp.int32)
    dst = edge_index[1].astype(jnp.int32)
    gs = gnn_edge_index[0].astype(jnp.int32)
    gd = gnn_edge_index[1].astype(jnp.int32)
    src2 = src.reshape(EG // C, C)
    dst2 = dst.reshape(EG // C, C)
    gs2 = gs.reshape(EN // C, C)
    gd2 = gd.reshape(EN // C, C)
    batT = batch.astype(jnp.int32).reshape(N, 1)

    gbt, gg = _tc_node0(
        posT, afT,
        lambda_sterics.reshape(1, 1), lambda_electrostatics.reshape(1, 1),
        Ws1, bs1.reshape(1, 32), Ws2.reshape(1, 32), bs2.reshape(1, 1),
        We1, be1.reshape(1, 32), We2.reshape(1, 32), be2.reshape(1, 1))

    g1 = _sc_gather_scalars(gbt.reshape(7 * N), src, (0, 1, 2, 3, 4),
                            dst, (0, 1, 2, 3), EG)
    d, I = _tc_gb1(g1)
    IpF = _sc_scatter_scalars([I], [dst2], [(0, 0, 0)], EG, 1)
    Ip = IpF.reshape(2, 1, NP)[:, :, :N]
    B, dBdI = _tc_node1(Ip, gbt)

    gnt = jnp.concatenate([B, afT[:4]], axis=0)
    X = _sc_gather_scalars(gnt.reshape(5 * N), gs, (0, 1, 2, 3, 4),
                           gd, (0, 1, 2, 3, 4), EN)
    m = _tc_l1(X, W1a, b1a.reshape(1, 64), W1b, b1b.reshape(1, 64))
    hp = _sc_scatter_rows(m, gd2, None)
    hpre, Ps, Pd = _tc_h(hp[:, 0], W2a[:64], W2a[64:])

    Asum = _sc_gather_rows_sum2(Ps, Pd, gs2, gd2)
    m2 = _tc_l2(Asum, b2a.reshape(1, 64), W2b)
    cpF = _sc_scatter_scalars(list(m2), [gd2], [(0, 0, 0), (1, 0, 1)], EN, 2)
    cp = cpF.reshape(2, 2, NP)[:, :, :N]
    Bc, parte, gcp1, s0, c0 = _tc_node2(cp, B, gbt, gg)

    qBc = jnp.concatenate([gbt[5:6], Bc], axis=0)
    g2 = _sc_gather_scalars(qBc.reshape(2 * N), src, (0, 1), dst, (0, 1), EG)
    e_pair, gBi, gBj, gdd = _tc_gb2(g2, d, gg)
    epF = _sc_scatter_scalars([e_pair, gBi, gBj], [dst2, src2],
                              [(0, 0, 0), (1, 0, 1), (2, 1, 2)], EG, 3)
    ep = epF.reshape(2, 3, NP)[:, :, :N]
    e_atom, gcp, gBdir = _tc_node3(ep, parte, gbt, Bc, B, s0, c0, gcp1, gg)

    gcpg = _sc_gather_scalars(gcp.reshape(2 * N), gd, (0, 1), None, None, EN)
    ga2 = _tc_l2b(Asum, gcpg, b2a.reshape(1, 64), W2b)
    Gp = _sc_scatter_rows(ga2, gs2, gd2)
    Rm = _tc_l1bn(Gp, hpre, W2a[:64].T, W2a[64:].T, W1b.T)
    Rg = _sc_gather_rows(Rm, gd2)
    Wb = jnp.concatenate([W1a[0:1], W1a[5:6]], axis=0)
    gB2 = _tc_l1b(X, Rg, W1a, b1a.reshape(1, 64), Wb)
    gpF = _sc_scatter_scalars(list(gB2), [gs2, gd2],
                              [(0, 0, 0), (1, 1, 1)], EN, 2)
    gp = gpF.reshape(2, 2, NP)[:, :, :N]
    gIsum = _tc_node4(gp, gBdir, dBdI)

    (gI,) = _sc_gather_scalars(gIsum.reshape(N), dst, (0,), None, None, EG)
    fv = _tc_force(g1, d, gdd, gI)
    fpF = _sc_scatter_scalars(
        list(fv), [dst2, src2],
        [(0, 0, 0), (1, 0, 1), (2, 0, 2), (0, 1, 3), (1, 1, 4), (2, 1, 5)],
        EG, 6)
    fp = fpF.reshape(2, 6, NP)[:, :, :N]
    F, en = _tc_final(fp, e_atom, batT)

    energy = en.reshape(NB, 1)
    forces = F.T
    return energy, forces


# speedup vs baseline: 49.0124x; 1.1031x over previous
import functools

import jax
import jax.numpy as jnp
from jax import lax
from jax.experimental import pallas as pl
from jax.experimental.pallas import tpu as pltpu
from jax.experimental.pallas import tpu_sc as plsc

N = 10000
EG = 640000
EN = 320000
NB = 64
FRACTION = 0.5
GAMMA = 0.00542
OFFSET = 0.0195141
COUL = 138.935485
EPS_FAC = 1.0 - 1.0 / 78.5
K2 = -0.5 * COUL * EPS_FAC

NP = 10240
CS = 2000
C = 80
BE_G = 20480
BE_N = 16384
BN = 2000
NC = 2
NS = 16
NW = NC * NS


def _silu(x):
    return x * jax.nn.sigmoid(x)


def _dsilu(x):
    s = jax.nn.sigmoid(x)
    return s * (1.0 + x * (1.0 - s))


def _pcall(body, **kw):
    return pl.pallas_call(body, **kw)


_SC_PARAMS = dict(
    compiler_params=pltpu.CompilerParams(use_tc_tiling_on_sc=False,
                                         needs_layout_passes=False))


def _sc_mesh():
    return plsc.VectorSubcoreMesh(core_axis_name="c", subcore_axis_name="s")


def _wid():
    return lax.axis_index("s") * NC + lax.axis_index("c")


def _sc_gather_scalars(tbl, idxA, rowsA, idxB, rowsB, E):
    T = tbl.shape[0] // N
    R = len(rowsA) + (len(rowsB) if rowsB is not None else 0)
    eper = E // NW
    nch = eper // CS

    @functools.partial(
        pl.kernel,
        mesh=_sc_mesh(),
        out_type=tuple(jax.ShapeDtypeStruct((E,), jnp.float32)
                       for _ in range(R)),
        scratch_types=[
            pltpu.VMEM((T * N,), jnp.float32),
            pltpu.VMEM((2, CS), jnp.int32),
            pltpu.VMEM((2, CS), jnp.int32),
            pltpu.VMEM((2, R, CS), jnp.float32),
            pltpu.SemaphoreType.DMA,
            pltpu.SemaphoreType.DMA,
            pltpu.SemaphoreType.DMA,
        ],
        **_SC_PARAMS,
    )
    def k(tbl_h, idxA_h, idxB_h, *rest):
        out_hs = rest[:R]
        tbl_v, ia_v, ib_v, out_v, s_t, s_l, s_o = rest[R:]
        wid = _wid()
        tcp = pltpu.async_copy(tbl_h, tbl_v, s_t)

        def load(b, buf):
            base = wid * eper + b * CS
            h = [pltpu.async_copy(idxA_h.at[pl.ds(base, CS)], ia_v.at[buf],
                                  s_l)]
            if rowsB is not None:
                h.append(pltpu.async_copy(idxB_h.at[pl.ds(base, CS)],
                                          ib_v.at[buf], s_l))
            return h

        pend_l = load(0, 0)
        tcp.wait()
        pend_o = [[], []]
        for b in range(nch):
            buf = b & 1
            for h in pend_l:
                h.wait()
            if b + 1 < nch:
                pend_l = load(b + 1, buf ^ 1)
            for h in pend_o[buf]:
                h.wait()

            def inner(i, carry, _buf=buf):
                for u in range(5):
                    off = i * 80 + u * 16
                    iva = ia_v[_buf, pl.ds(off, 16)]
                    r = 0
                    for row in rowsA:
                        v = plsc.load_gather(tbl_v, [iva + row * N])
                        out_v[_buf, r, pl.ds(off, 16)] = v
                        r += 1
                    if rowsB is not None:
                        ivb = ib_v[_buf, pl.ds(off, 16)]
                        for row in rowsB:
                            v = plsc.load_gather(tbl_v, [ivb + row * N])
                            out_v[_buf, r, pl.ds(off, 16)] = v
                            r += 1
                return carry

            lax.fori_loop(0, CS // 80, inner, 0)
            base = wid * eper + b * CS
            pend_o[buf] = [
                pltpu.async_copy(out_v.at[buf, r],
                                 out_hs[r].at[pl.ds(base, CS)], s_o)
                for r in range(R)]
        for hb in pend_o:
            for h in hb:
                h.wait()

    idxB_arg = idxB if idxB is not None else idxA
    return k(tbl, idxA, idxB_arg)


def _sc_scatter_scalars(vals, idxs, specs, E, nslots):
    eper = E // NW
    BK = 2000 if eper % 2000 == 0 else 400
    W = BK // C
    nblk = eper // BK
    ACC = nslots * NP
    stripe = ACC // NS
    nv = len(vals)
    ni = len(idxs)
    ns = len(specs)

    @functools.partial(
        pl.kernel,
        mesh=_sc_mesh(),
        out_type=jax.ShapeDtypeStruct((2, ACC), jnp.float32),
        scratch_types=[
            pltpu.VMEM_SHARED((ACC,), jnp.float32),
            pltpu.VMEM((stripe,), jnp.float32),
            pltpu.VMEM((ni, W, C), jnp.int32),
            pltpu.VMEM((ns, W, C), jnp.int32),
            pltpu.VMEM((ns, BK), jnp.float32),
            pltpu.SemaphoreType.DMA,
            pltpu.SemaphoreType.DMA,
        ],
        **_SC_PARAMS,
    )
    def k(*args):
        vals_hs = args[:nv]
        idx_hs = args[nv:nv + ni]
        out_h = args[nv + ni]
        acc, zb, iv, io, vv, s_l, s_s = args[nv + ni + 1:]
        cc = lax.axis_index("c")
        sid = lax.axis_index("s")
        wid = _wid()

        def zloop(i, carry):
            zb[pl.ds(i * 16, 16)] = jnp.zeros((16,), jnp.float32)
            return carry

        lax.fori_loop(0, stripe // 16, zloop, 0)
        pltpu.sync_copy(zb, acc.at[pl.ds(sid * stripe, stripe)])
        plsc.subcore_barrier()

        def blk(b, carry):
            irow = wid * (eper // C) + b * W
            hs = []
            for t in range(ni):
                hs.append(pltpu.async_copy(idx_hs[t].at[pl.ds(irow, W)],
                                           iv.at[t], s_l))
            for si, (vr, ii, slot) in enumerate(specs):
                hs.append(pltpu.async_copy(
                    vals_hs[vr].at[pl.ds(wid * eper + b * BK, BK)],
                    vv.at[si], s_l))
            for h in hs:
                h.wait()
            for si, (vr, ii, slot) in enumerate(specs):
                def offl(i, carry2, _si=si, _ii=ii, _slot=slot):
                    for w in range(W):
                        io[_si, w, pl.ds(i * 16, 16)] = (
                            iv[_ii, w, pl.ds(i * 16, 16)] + _slot * NP)
                    return carry2

                lax.fori_loop(0, C // 16, offl, 0)
            ss = []
            for si in range(ns):
                for w in range(W):
                    ss.append(pltpu.async_copy(
                        vv.at[si, pl.ds(w * C, C)], acc.at[io.at[si, w]],
                        s_s, add=True))
            for h in ss:
                h.wait()
            return carry

        lax.fori_loop(0, nblk, blk, 0)
        plsc.subcore_barrier()
        pltpu.sync_copy(acc.at[pl.ds(sid * stripe, stripe)],
                        out_h.at[cc, pl.ds(sid * stripe, stripe)])

    return k(*vals, *idxs)


def _sc_scatter_rows(vals, idxA2, idxB2):
    eper = EN // NW
    BK = 400
    W = BK // C
    nblk = eper // BK
    dual = idxB2 is not None
    nsets = 2 if dual else 1
    rstripe = N // NS

    scratch = [
        pltpu.VMEM_SHARED((N, 64), jnp.float32),
        pltpu.VMEM((rstripe // 5, 64), jnp.float32),
        pltpu.VMEM((nsets, W, C), jnp.int32),
        pltpu.VMEM((BK, 64), jnp.float32),
        pltpu.SemaphoreType.DMA,
        pltpu.SemaphoreType.DMA,
    ]
    if dual:
        scratch.insert(1, pltpu.VMEM_SHARED((N, 64), jnp.float32))

    @functools.partial(
        pl.kernel,
        mesh=_sc_mesh(),
        out_type=jax.ShapeDtypeStruct((2, nsets, N, 64), jnp.float32),
        scratch_types=scratch,
        **_SC_PARAMS,
    )
    def k(vals_h, idxA_h, idxB_h, out_h, *scr):
        if dual:
            accA, accB, zb, iv, vv, s_l, s_s = scr
            accs = (accA, accB)
            idx_hs = (idxA_h, idxB_h)
        else:
            accA, zb, iv, vv, s_l, s_s = scr
            accs = (accA,)
            idx_hs = (idxA_h,)
        cc = lax.axis_index("c")
        sid = lax.axis_index("s")
        wid = _wid()

        def zloop(i, carry):
            for c4 in range(4):
                zb[i, pl.ds(c4 * 16, 16)] = jnp.zeros((16,), jnp.float32)
            return carry

        lax.fori_loop(0, rstripe // 5, zloop, 0)
        for a in accs:
            for p in range(5):
                pltpu.sync_copy(
                    zb, a.at[pl.ds(sid * rstripe + p * (rstripe // 5),
                                   rstripe // 5), :])
        plsc.subcore_barrier()

        def blk(b, carry):
            irow = wid * (eper // C) + b * W
            hs = [pltpu.async_copy(vals_h.at[pl.ds(wid * eper + b * BK, BK), :],
                                   vv, s_l)]
            for t in range(nsets):
                hs.append(pltpu.async_copy(idx_hs[t].at[pl.ds(irow, W)],
                                           iv.at[t], s_l))
            for h in hs:
                h.wait()
            ss = []
            for t in range(nsets):
                for w in range(W):
                    ss.append(pltpu.async_copy(
                        vv.at[pl.ds(w * C, C), :], accs[t].at[iv.at[t, w]],
                        s_s, add=True))
            for h in ss:
                h.wait()
            return carry

        lax.fori_loop(0, nblk, blk, 0)
        plsc.subcore_barrier()
        for t in range(nsets):
            pltpu.sync_copy(accs[t].at[pl.ds(sid * rstripe, rstripe), :],
                            out_h.at[cc, t, pl.ds(sid * rstripe, rstripe), :])

    idxB_arg = idxB2 if dual else idxA2
    return k(vals, idxA2, idxB_arg)


def _sc_gather_rows_sum2(Ps, Pd, gs2, gd2):
    eper = EN // NW
    BK = 400
    W = BK // C
    nblk = eper // BK

    @functools.partial(
        pl.kernel,
        mesh=_sc_mesh(),
        out_type=jax.ShapeDtypeStruct((EN, 64), jnp.float32),
        scratch_types=[
            pltpu.VMEM((2, W, C), jnp.int32),
            pltpu.VMEM((2, BK, 64), jnp.float32),
            pltpu.VMEM((2, BK, 64), jnp.float32),
            pltpu.SemaphoreType.DMA,
            pltpu.SemaphoreType.DMA,
            pltpu.SemaphoreType.DMA,
        ],
        **_SC_PARAMS,
    )
    def k(Ps_h, Pd_h, gs_h, gd_h, out_h, iv, ra, rb, s_l, s_g, s_o):
        wid = _wid()
        pend_o = [None, None]
        for b in range(nblk):
            buf = b & 1
            irow = wid * (eper // C) + b * W
            h1 = pltpu.async_copy(gs_h.at[pl.ds(irow, W)], iv.at[0], s_l)
            h2 = pltpu.async_copy(gd_h.at[pl.ds(irow, W)], iv.at[1], s_l)
            h1.wait()
            h2.wait()
            if pend_o[buf] is not None:
                pend_o[buf].wait()
            gh = []
            for w in range(W):
                gh.append(pltpu.async_copy(
                    Ps_h.at[iv.at[0, w]],
                    ra.at[buf, pl.ds(w * C, C), :], s_g))
                gh.append(pltpu.async_copy(
                    Pd_h.at[iv.at[1, w]],
                    rb.at[buf, pl.ds(w * C, C), :], s_g))
            for h in gh:
                h.wait()

            def addl(i, carry2, _buf=buf):
                for c4 in range(4):
                    ra[_buf, i, pl.ds(c4 * 16, 16)] = (
                        ra[_buf, i, pl.ds(c4 * 16, 16)]
                        + rb[_buf, i, pl.ds(c4 * 16, 16)])
                return carry2

            lax.fori_loop(0, BK, addl, 0)
            pend_o[buf] = pltpu.async_copy(
                ra.at[buf], out_h.at[pl.ds(wid * eper + b * BK, BK), :], s_o)
        for h in pend_o:
            if h is not None:
                h.wait()

    return k(Ps, Pd, gs2, gd2)


def _sc_gather_rows(tab, idx2):
    eper = EN // NW
    BK = 400
    W = BK // C
    nblk = eper // BK

    @functools.partial(
        pl.kernel,
        mesh=_sc_mesh(),
        out_type=jax.ShapeDtypeStruct((EN, 64), jnp.float32),
        scratch_types=[
            pltpu.VMEM((W, C), jnp.int32),
            pltpu.VMEM((2, BK, 64), jnp.float32),
            pltpu.SemaphoreType.DMA,
            pltpu.SemaphoreType.DMA,
            pltpu.SemaphoreType.DMA,
        ],
        **_SC_PARAMS,
    )
    def k(tab_h, idx_h, out_h, iv, ra, s_l, s_g, s_o):
        wid = _wid()
        pend_o = [None, None]
        for b in range(nblk):
            buf = b & 1
            irow = wid * (eper // C) + b * W
            pltpu.async_copy(idx_h.at[pl.ds(irow, W)], iv, s_l).wait()
            if pend_o[buf] is not None:
                pend_o[buf].wait()
            gh = []
            for w in range(W):
                gh.append(pltpu.async_copy(
                    tab_h.at[iv.at[w]], ra.at[buf, pl.ds(w * C, C), :], s_g))
            for h in gh:
                h.wait()
            pend_o[buf] = pltpu.async_copy(
                ra.at[buf], out_h.at[pl.ds(wid * eper + b * BK, BK), :], s_o)
        for h in pend_o:
            if h is not None:
                h.wait()

    return k(tab, idx2)


def _espec(BE):
    return pl.BlockSpec((BE,), lambda i: (i,))


def _eshape(E):
    return jax.ShapeDtypeStruct((E,), jnp.float32)


def _tc_node0(posT, afT, lamS, lamE, Ws1, bs1, Ws2T, bs2, We1, be1, We2T, be2):
    def body(posT_r, afT_r, lamS_r, lamE_r, Ws1_r, bs1_r, Ws2T_r, bs2_r,
             We1_r, be1_r, We2T_r, be2_r, gbt_r, gg_r):
        p = posT_r[...]
        af = afT_r[...]
        af0 = af[0:1]
        af1 = af[1:2]
        af2 = af[2:3]
        q = af0 - 0.5
        rho = 0.1 + 0.1 * af1
        scale = 0.8 + 0.4 * af2
        or_ = rho - OFFSET
        sa = GAMMA * (rho + 0.14) ** 2
        gbt_r[...] = jnp.concatenate([p, or_, scale, q, sa], axis=0)
        gsv = jax.nn.sigmoid(
            jnp.sum(_silu(lamS_r[...] * Ws1_r[...] + bs1_r[...]) * Ws2T_r[...],
                    axis=1, keepdims=True) + bs2_r[...])
        gev = jax.nn.sigmoid(
            jnp.sum(_silu(lamE_r[...] * We1_r[...] + be1_r[...]) * We2T_r[...],
                    axis=1, keepdims=True) + be2_r[...])
        gg_r[...] = jnp.concatenate([gsv, gev], axis=1)

    return _pcall(
        body,
        out_shape=(jax.ShapeDtypeStruct((7, N), jnp.float32),
                   jax.ShapeDtypeStruct((1, 2), jnp.float32)),
    )(posT, afT, lamS, lamE, Ws1, bs1, Ws2T, bs2, We1, be1, We2T, be2)


def _tc_gb1(g1):
    def body(xs, ys, zs, orj, scj, xd, yd, zd, ori, d_r, I_r):
        ddx = xs[...] - xd[...]
        ddy = ys[...] - yd[...]
        ddz = zs[...] - zd[...]
        or_i = ori[...]
        d = jnp.sqrt(ddx * ddx + ddy * ddy + ddz * ddz + 1e-12)
        sr = scj[...] * orj[...]
        L = jnp.maximum(jnp.abs(d - sr), or_i)
        U = d + sr
        I = 0.5 * (1.0 / L - 1.0 / U
                   + 0.25 * (d - sr * sr / d) * (1.0 / (U * U) - 1.0 / (L * L))
                   + 0.5 * jnp.log(L / U) / d)
        mask = (or_i < U).astype(jnp.float32)
        d_r[...] = d
        I_r[...] = I * mask

    nb = pl.cdiv(EG, BE_G)
    sp = _espec(BE_G)
    return _pcall(
        body,
        grid=(nb,),
        in_specs=[sp] * 9,
        out_specs=(sp, sp),
        out_shape=(_eshape(EG), _eshape(EG)),
    )(*g1)


def _tc_node1(Ip, gbt):
    def body(Ip_r, gbt_r, B_r, dBdI_r):
        Isum = Ip_r[0, 0:1] + Ip_r[1, 0:1]
        or_ = gbt_r[3:4]
        x = 1.0 / or_ - Isum
        xc = jnp.clip(x, 0.5, 200.0)
        B = 1.0 / xc
        mask = ((x > 0.5) & (x < 200.0)).astype(jnp.float32)
        B_r[...] = B
        dBdI_r[...] = B * B * mask

    return _pcall(
        body,
        out_shape=(jax.ShapeDtypeStruct((1, N), jnp.float32),
                   jax.ShapeDtypeStruct((1, N), jnp.float32)),
    )(Ip, gbt)


def _tc_l1(X, W1a, b1a, W1b, b1b):
    def body(*args):
        xs = args[:10]
        W1a_r, b1a_r, W1b_r, b1b_r, m_r = args[10:]
        Xb = jnp.concatenate([x[...].reshape(1, BE_N) for x in xs], axis=0)
        a1 = lax.dot_general(Xb, W1a_r[...], (((0,), (0,)), ((), ())),
                             preferred_element_type=jnp.float32) + b1a_r[...]
        m_r[...] = lax.dot_general(_silu(a1), W1b_r[...],
                                   (((1,), (0,)), ((), ())),
                                   preferred_element_type=jnp.float32) + b1b_r[...]

    nb = pl.cdiv(EN, BE_N)
    sp = _espec(BE_N)
    return _pcall(
        body,
        grid=(nb,),
        in_specs=[sp] * 10 + [
            pl.BlockSpec((10, 64), lambda i: (0, 0)),
            pl.BlockSpec((1, 64), lambda i: (0, 0)),
            pl.BlockSpec((64, 64), lambda i: (0, 0)),
            pl.BlockSpec((1, 64), lambda i: (0, 0)),
        ],
        out_specs=pl.BlockSpec((BE_N, 64), lambda i: (i, 0)),
        out_shape=jax.ShapeDtypeStruct((EN, 64), jnp.float32),
    )(*X, W1a, b1a, W1b, b1b)


def _tc_h(hp, W2as, W2ad):
    def body(hp_r, W2as_r, W2ad_r, hpre_r, Ps_r, Pd_r):
        hpre = hp_r[0] + hp_r[1]
        h = _silu(hpre)
        hpre_r[...] = hpre
        Ps_r[...] = jnp.dot(h, W2as_r[...], preferred_element_type=jnp.float32)
        Pd_r[...] = jnp.dot(h, W2ad_r[...], preferred_element_type=jnp.float32)

    nb = N // BN
    return _pcall(
        body,
        grid=(nb,),
        in_specs=[
            pl.BlockSpec((2, BN, 64), lambda i: (0, i, 0)),
            pl.BlockSpec((64, 64), lambda i: (0, 0)),
            pl.BlockSpec((64, 64), lambda i: (0, 0)),
        ],
        out_specs=(pl.BlockSpec((BN, 64), lambda i: (i, 0)),
                   pl.BlockSpec((BN, 64), lambda i: (i, 0)),
                   pl.BlockSpec((BN, 64), lambda i: (i, 0))),
        out_shape=(jax.ShapeDtypeStruct((N, 64), jnp.float32),
                   jax.ShapeDtypeStruct((N, 64), jnp.float32),
                   jax.ShapeDtypeStruct((N, 64), jnp.float32)),
    )(hp, W2as, W2ad)


def _tc_l2(Asum, b2a, W2b):
    def body(A_r, b2a_r, W2b_r, m20_r, m21_r):
        sa2 = _silu(A_r[...] + b2a_r[...])
        m2 = lax.dot_general(W2b_r[...], sa2, (((0,), (1,)), ((), ())),
                             preferred_element_type=jnp.float32)
        m20_r[...] = m2[0]
        m21_r[...] = m2[1]

    nb = pl.cdiv(EN, BE_N)
    sp = _espec(BE_N)
    return _pcall(
        body,
        grid=(nb,),
        in_specs=[
            pl.BlockSpec((BE_N, 64), lambda i: (i, 0)),
            pl.BlockSpec((1, 64), lambda i: (0, 0)),
            pl.BlockSpec((64, 2), lambda i: (0, 0)),
        ],
        out_specs=(sp, sp),
        out_shape=(_eshape(EN), _eshape(EN)),
    )(Asum, b2a, W2b)


def _tc_node2(cp, B, gbt, gg):
    def body(cp_r, B_r, gbt_r, gg_r, Bc_r, parte_r, gcp1_r, s0_r, c0_r):
        cpre0 = cp_r[0, 0:1] + cp_r[1, 0:1]
        cpre1 = cp_r[0, 1:2] + cp_r[1, 1:2]
        c0 = jax.nn.sigmoid(cpre0)
        c1 = jax.nn.sigmoid(cpre1)
        B = B_r[...]
        q = gbt_r[5:6]
        sa = gbt_r[6:7]
        gsv = gg_r[0:1, 0:1]
        gev = gg_r[0:1, 1:2]
        Bc = B * (FRACTION * c0 + (1.0 - FRACTION))
        e_self = K2 * q * q / Bc
        Bc_r[...] = Bc
        parte_r[...] = e_self * gev + sa * c1 * gsv
        gcp1_r[...] = sa * gsv * c1 * (1.0 - c1)
        s0_r[...] = c0 * (1.0 - c0)
        c0_r[...] = c0

    shp = jax.ShapeDtypeStruct((1, N), jnp.float32)
    return _pcall(
        body,
        out_shape=(shp, shp, shp, shp, shp),
    )(cp, B, gbt, gg)


def _tc_gb2(g2, d, gg):
    def body(qs, Bcs, qd, Bcd, d_r, gg_r, ep_r, gBi_r, gBj_r, gdd_r):
        q_s = qs[...]
        Bc_s = Bcs[...]
        q_d = qd[...]
        Bc_d = Bcd[...]
        d = d_r[...]
        gev = gg_r[0, 1]
        u = Bc_d * Bc_s
        ex = jnp.exp(-(d * d) / (4.0 * u))
        f2 = d * d + u * ex
        f = jnp.sqrt(f2)
        w = K2 * q_d * q_s
        e_pair = w / f
        dedf = -w / f2
        dfdd = (2.0 * d - 0.5 * d * ex) / (2.0 * f)
        dfdu = ex * (1.0 + d * d / (4.0 * u)) / (2.0 * f)
        gu = gev * dedf * dfdu
        ep_r[...] = e_pair
        gBi_r[...] = gu * Bc_s
        gBj_r[...] = gu * Bc_d
        gdd_r[...] = gev * dedf * dfdd

    nb = pl.cdiv(EG, BE_G)
    sp = _espec(BE_G)
    return _pcall(
        body,
        grid=(nb,),
        in_specs=[sp] * 5 + [pl.BlockSpec((1, 2), lambda i: (0, 0))],
        out_specs=(sp, sp, sp, sp),
        out_shape=(_eshape(EG),) * 4,
    )(*g2, d, gg)


def _tc_node3(ep, parte, gbt, Bc, B, s0, c0, gcp1, gg):
    def body(ep_r, parte_r, gbt_r, Bc_r, B_r, s0_r, c0_r, gcp1_r, gg_r,
             ea_r, gcp_r, gBdir_r):
        e_gb = ep_r[0, 0:1] + ep_r[1, 0:1]
        gBi_n = ep_r[0, 1:2] + ep_r[1, 1:2]
        gBj_n = ep_r[0, 2:3] + ep_r[1, 2:3]
        q = gbt_r[5:6]
        gev = gg_r[0:1, 1:2]
        Bc = Bc_r[...]
        B = B_r[...]
        s0 = s0_r[...]
        c0 = c0_r[...]
        ea_r[...] = parte_r[...] + e_gb * gev
        gBc = gBi_n + gBj_n - gev * K2 * q * q / (Bc * Bc)
        gcp0 = gBc * B * FRACTION * s0
        gcp_r[...] = jnp.concatenate([gcp0, gcp1_r[...]], axis=0)
        gBdir_r[...] = gBc * (FRACTION * c0 + (1.0 - FRACTION))

    return _pcall(
        body,
        out_shape=(jax.ShapeDtypeStruct((1, N), jnp.float32),
                   jax.ShapeDtypeStruct((2, N), jnp.float32),
                   jax.ShapeDtypeStruct((1, N), jnp.float32)),
    )(ep, parte, gbt, Bc, B, s0, c0, gcp1, gg)


def _tc_l2b(Asum, gcpg, b2a, W2b):
    def body(A_r, g0_r, g1_r, b2a_r, W2b_r, ga2_r):
        a2 = A_r[...] + b2a_r[...]
        gm = jnp.concatenate([g0_r[...].reshape(1, BE_N),
                              g1_r[...].reshape(1, BE_N)], axis=0)
        gsa2 = lax.dot_general(gm, W2b_r[...], (((0,), (1,)), ((), ())),
                               preferred_element_type=jnp.float32)
        ga2_r[...] = gsa2 * _dsilu(a2)

    nb = pl.cdiv(EN, BE_N)
    sp = _espec(BE_N)
    return _pcall(
        body,
        grid=(nb,),
        in_specs=[
            pl.BlockSpec((BE_N, 64), lambda i: (i, 0)),
            sp, sp,
            pl.BlockSpec((1, 64), lambda i: (0, 0)),
            pl.BlockSpec((64, 2), lambda i: (0, 0)),
        ],
        out_specs=pl.BlockSpec((BE_N, 64), lambda i: (i, 0)),
        out_shape=jax.ShapeDtypeStruct((EN, 64), jnp.float32),
    )(Asum, gcpg[0], gcpg[1], b2a, W2b)


def _tc_l1bn(Gp, hpre, W2asT, W2adT, W1bT):
    def body(Gp_r, hpre_r, W2asT_r, W2adT_r, W1bT_r, R_r):
        Gs = Gp_r[0, 0] + Gp_r[1, 0]
        Gd = Gp_r[0, 1] + Gp_r[1, 1]
        gh = (jnp.dot(Gs, W2asT_r[...], preferred_element_type=jnp.float32)
              + jnp.dot(Gd, W2adT_r[...], preferred_element_type=jnp.float32))
        ghp = gh * _dsilu(hpre_r[...])
        R_r[...] = jnp.dot(ghp, W1bT_r[...], preferred_element_type=jnp.float32)

    nb = N // BN
    return _pcall(
        body,
        grid=(nb,),
        in_specs=[
            pl.BlockSpec((2, 2, BN, 64), lambda i: (0, 0, i, 0)),
            pl.BlockSpec((BN, 64), lambda i: (i, 0)),
            pl.BlockSpec((64, 64), lambda i: (0, 0)),
            pl.BlockSpec((64, 64), lambda i: (0, 0)),
            pl.BlockSpec((64, 64), lambda i: (0, 0)),
        ],
        out_specs=pl.BlockSpec((BN, 64), lambda i: (i, 0)),
        out_shape=jax.ShapeDtypeStruct((N, 64), jnp.float32),
    )(Gp, hpre, W2asT, W2adT, W1bT)


def _tc_l1b(X, Rg, W1a, b1a, Wb):
    def body(*args):
        xs = args[:10]
        Rg_r, W1a_r, b1a_r, Wb_r, gBs_r, gBd_r = args[10:]
        Xb = jnp.concatenate([x[...].reshape(1, BE_N) for x in xs], axis=0)
        a1 = lax.dot_general(Xb, W1a_r[...], (((0,), (0,)), ((), ())),
                             preferred_element_type=jnp.float32) + b1a_r[...]
        ga1 = Rg_r[...] * _dsilu(a1)
        gB = lax.dot_general(Wb_r[...], ga1, (((1,), (1,)), ((), ())),
                             preferred_element_type=jnp.float32)
        gBs_r[...] = gB[0]
        gBd_r[...] = gB[1]

    nb = pl.cdiv(EN, BE_N)
    sp = _espec(BE_N)
    return _pcall(
        body,
        grid=(nb,),
        in_specs=[sp] * 10 + [
            pl.BlockSpec((BE_N, 64), lambda i: (i, 0)),
            pl.BlockSpec((10, 64), lambda i: (0, 0)),
            pl.BlockSpec((1, 64), lambda i: (0, 0)),
            pl.BlockSpec((2, 64), lambda i: (0, 0)),
        ],
        out_specs=(sp, sp),
        out_shape=(_eshape(EN), _eshape(EN)),
    )(*X, Rg, W1a, b1a, Wb)


def _tc_node4(gp, gBdir, dBdI):
    def body(gp_r, gBdir_r, dBdI_r, gI_r):
        gB = gp_r[0, 0:1] + gp_r[1, 0:1] + gp_r[0, 1:2] + gp_r[1, 1:2]
        gI_r[...] = (gBdir_r[...] + gB) * dBdI_r[...]

    return _pcall(
        body,
        out_shape=jax.ShapeDtypeStruct((1, N), jnp.float32),
    )(gp, gBdir, dBdI)


def _tc_force(g1, d, gdd, gI):
    def body(xs, ys, zs, orj, scj, xd, yd, zd, ori, d_r, gdd_r, gI_r,
             fx_r, fy_r, fz_r):
        ddx = xs[...] - xd[...]
        ddy = ys[...] - yd[...]
        ddz = zs[...] - zd[...]
        or_i = ori[...]
        d = d_r[...]
        sr = scj[...] * orj[...]
        L = jnp.maximum(jnp.abs(d - sr), or_i)
        U = d + sr
        mask = (or_i < U).astype(jnp.float32)
        absds = jnp.abs(d - sr)
        dLdd = jnp.sign(d - sr) * (absds > or_i).astype(jnp.float32)
        iL = 1.0 / L
        iU = 1.0 / U
        idd = 1.0 / d
        t = d - sr * sr * idd
        dIdL = 0.5 * (-iL * iL + 0.5 * t * iL * iL * iL + 0.5 * iL * idd)
        dIdU = 0.5 * (iU * iU - 0.5 * t * iU * iU * iU - 0.5 * iU * idd)
        dIdd_exp = 0.5 * (0.25 * (1.0 + (sr * idd) ** 2) * (iU * iU - iL * iL)
                          - 0.5 * jnp.log(L * iU) * idd * idd)
        dIdd = (dIdL * dLdd + dIdU + dIdd_exp) * mask
        g_tot = gdd_r[...] + gI_r[...] * dIdd
        coef = g_tot * idd
        fx_r[...] = coef * ddx
        fy_r[...] = coef * ddy
        fz_r[...] = coef * ddz

    nb = pl.cdiv(EG, BE_G)
    sp = _espec(BE_G)
    return _pcall(
        body,
        grid=(nb,),
        in_specs=[sp] * 12,
        out_specs=(sp, sp, sp),
        out_shape=(_eshape(EG),) * 3,
    )(*g1, d, gdd, gI)


def _tc_final(fp, e_atom, batT):
    def body(fp_r, ea_r, bat_r, F_r, en_r):
        F_r[...] = ((fp_r[0, 0:3] + fp_r[1, 0:3])
                    - (fp_r[0, 3:6] + fp_r[1, 3:6]))
        oh = (bat_r[...] == lax.broadcasted_iota(jnp.int32, (N, NB), 1)
              ).astype(jnp.float32)
        en_r[...] = lax.dot_general(ea_r[...], oh, (((1,), (0,)), ((), ())),
                                    preferred_element_type=jnp.float32)

    return _pcall(
        body,
        out_shape=(jax.ShapeDtypeStruct((3, N), jnp.float32),
                   jax.ShapeDtypeStruct((1, NB), jnp.float32)),
    )(fp, e_atom, batT)


def kernel(positions, atom_features, lambda_sterics, lambda_electrostatics,
           retrieve_forces, batch, edge_index, gnn_edge_index,
           W1a, b1a, W1b, b1b, W2a, b2a, W2b, b2b,
           Ws1, bs1, Ws2, bs2, We1, be1, We2, be2):
    posT = positions.T
    afT = atom_features.T
    src = edge_index[0].astype(jnp.int32)
    dst = edge_index[1].astype(jnp.int32)
    gs = gnn_edge_index[0].astype(jnp.int32)
    gd = gnn_edge_index[1].astype(jnp.int32)
    src2 = src.reshape(EG // C, C)
    dst2 = dst.reshape(EG // C, C)
    gs2 = gs.reshape(EN // C, C)
    gd2 = gd.reshape(EN // C, C)
    batT = batch.astype(jnp.int32).reshape(N, 1)

    gbt, gg = _tc_node0(
        posT, afT,
        lambda_sterics.reshape(1, 1), lambda_electrostatics.reshape(1, 1),
        Ws1, bs1.reshape(1, 32), Ws2.reshape(1, 32), bs2.reshape(1, 1),
        We1, be1.reshape(1, 32), We2.reshape(1, 32), be2.reshape(1, 1))

    g1 = _sc_gather_scalars(gbt.reshape(7 * N), src, (0, 1, 2, 3, 4),
                            dst, (0, 1, 2, 3), EG)
    d, I = _tc_gb1(g1)
    IpF = _sc_scatter_scalars([I], [dst2], [(0, 0, 0)], EG, 1)
    Ip = IpF.reshape(2, 1, NP)[:, :, :N]
    B, dBdI = _tc_node1(Ip, gbt)

    gnt = jnp.concatenate([B, afT[:4]], axis=0)
    X = _sc_gather_scalars(gnt.reshape(5 * N), gs, (0, 1, 2, 3, 4),
                           gd, (0, 1, 2, 3, 4), EN)
    m = _tc_l1(X, W1a, b1a.reshape(1, 64), W1b, b1b.reshape(1, 64))
    hp = _sc_scatter_rows(m, gd2, None)
    hpre, Ps, Pd = _tc_h(hp[:, 0], W2a[:64], W2a[64:])

    Asum = _sc_gather_rows_sum2(Ps, Pd, gs2, gd2)
    m2 = _tc_l2(Asum, b2a.reshape(1, 64), W2b)
    cpF = _sc_scatter_scalars(list(m2), [gd2], [(0, 0, 0), (1, 0, 1)], EN, 2)
    cp = cpF.reshape(2, 2, NP)[:, :, :N]
    Bc, parte, gcp1, s0, c0 = _tc_node2(cp, B, gbt, gg)

    qBc = jnp.concatenate([gbt[5:6], Bc], axis=0)
    g2 = _sc_gather_scalars(qBc.reshape(2 * N), src, (0, 1), dst, (0, 1), EG)
    e_pair, gBi, gBj, gdd = _tc_gb2(g2, d, gg)
    epF = _sc_scatter_scalars([e_pair, gBi, gBj], [dst2, src2],
                              [(0, 0, 0), (1, 0, 1), (2, 1, 2)], EG, 3)
    ep = epF.reshape(2, 3, NP)[:, :, :N]
    e_atom, gcp, gBdir = _tc_node3(ep, parte, gbt, Bc, B, s0, c0, gcp1, gg)

    gcpg = _sc_gather_scalars(gcp.reshape(2 * N), gd, (0, 1), None, None, EN)
    ga2 = _tc_l2b(Asum, gcpg, b2a.reshape(1, 64), W2b)
    Gp = _sc_scatter_rows(ga2, gs2, gd2)
    Rm = _tc_l1bn(Gp, hpre, W2a[:64].T, W2a[64:].T, W1b.T)
    Rg = _sc_gather_rows(Rm, gd2)
    Wb = jnp.concatenate([W1a[0:1], W1a[5:6]], axis=0)
    gB2 = _tc_l1b(X, Rg, W1a, b1a.reshape(1, 64), Wb)
    gpF = _sc_scatter_scalars(list(gB2), [gs2, gd2],
                              [(0, 0, 0), (1, 1, 1)], EN, 2)
    gp = gpF.reshape(2, 2, NP)[:, :, :N]
    gIsum = _tc_node4(gp, gBdir, dBdI)

    (gI,) = _sc_gather_scalars(gIsum.reshape(N), dst, (0,), None, None, EG)
    fv = _tc_force(g1, d, gdd, gI)
    fpF = _sc_scatter_scalars(
        list(fv), [dst2, src2],
        [(0, 0, 0), (1, 0, 1), (2, 0, 2), (0, 1, 3), (1, 1, 4), (2, 1, 5)],
        EG, 6)
    fp = fpF.reshape(2, 6, NP)[:, :, :N]
    F, en = _tc_final(fp, e_atom, batT)

    energy = en.reshape(NB, 1)
    forces = F.T
    return energy, forces
